# baseline (speedup 1.0000x reference)
import functools

import jax
import jax.numpy as jnp
from jax.experimental import pallas as pl
from jax.experimental.pallas import tpu as pltpu


_BR = 1024


def _act(x, act):
    if act == "leaky":
        return jnp.where(x >= 0, x, 0.01 * x)
    if act == "relu":
        return jnp.maximum(x, 0.0)
    return x


def _mm1_body(a_ref, w_ref, b_ref, o_ref, *, act):
    x = jnp.dot(a_ref[...], w_ref[...], preferred_element_type=jnp.float32)
    o_ref[...] = _act(x + b_ref[...], act)


def _mm2_body(a_ref, b2_ref, w1_ref, w2_ref, b_ref, o_ref, *, act):
    x = jnp.dot(a_ref[...], w1_ref[...], preferred_element_type=jnp.float32)
    x = x + jnp.dot(b2_ref[...], w2_ref[...], preferred_element_type=jnp.float32)
    o_ref[...] = _act(x + b_ref[...], act)


def _mm(A, W, b, act=None):
    n, k = A.shape
    ko, m = W.shape
    grid = (pl.cdiv(n, _BR),)
    return pl.pallas_call(
        functools.partial(_mm1_body, act=act),
        grid=grid,
        in_specs=[
            pl.BlockSpec((_BR, k), lambda i: (i, 0)),
            pl.BlockSpec((ko, m), lambda i: (0, 0)),
            pl.BlockSpec((1, m), lambda i: (0, 0)),
        ],
        out_specs=pl.BlockSpec((_BR, m), lambda i: (i, 0)),
        out_shape=jax.ShapeDtypeStruct((n, m), jnp.float32),
    )(A, W, b.reshape(1, m))


def _mm2(A, B, W1, W2, b, act=None):
    n, k = A.shape
    m = W1.shape[1]
    grid = (pl.cdiv(n, _BR),)
    return pl.pallas_call(
        functools.partial(_mm2_body, act=act),
        grid=grid,
        in_specs=[
            pl.BlockSpec((_BR, k), lambda i: (i, 0)),
            pl.BlockSpec((_BR, B.shape[1]), lambda i: (i, 0)),
            pl.BlockSpec((W1.shape[0], m), lambda i: (0, 0)),
            pl.BlockSpec((W2.shape[0], m), lambda i: (0, 0)),
            pl.BlockSpec((1, m), lambda i: (0, 0)),
        ],
        out_specs=pl.BlockSpec((_BR, m), lambda i: (i, 0)),
        out_shape=jax.ShapeDtypeStruct((n, m), jnp.float32),
    )(A, B, W1, W2, b.reshape(1, m))


def _graph_stage(h, ss, sd, inz, iz, lam, W_c1, b_c1, W_c2, b_c2,
                 W_sg1, b_sg1, W_sg2, b_sg2, W_lin3, b_lin3):
    n = inz.shape[0]
    D = h.shape[1]
    deg = jnp.zeros((n,), jnp.float32).at[sd].add(1.0)
    dinv = jnp.clip(deg, 1.0, None) ** -0.5
    r = 2.0 / lam
    h_nz = h[inz]
    h_z = h[iz]

    def cheb(X0, W, b):
        xs = X0 * dinv[:, None]
        msg = jnp.zeros((n, D), jnp.float32).at[sd].add(xs[ss])
        X1 = (r * dinv)[:, None] * msg - X0
        return _mm2(X0, X1, W[:D], W[D:], b, act="relu")

    h01 = cheb(h_nz, W_c1, b_c1)
    h11 = cheb(h01, W_c2, b_c2)
    top = _mm2(h01, h11, W_lin3[:D], W_lin3[D:], b_lin3)
    h02 = _mm(h_z, W_sg1, b_sg1)
    h12 = _mm(h02, W_sg2, b_sg2)
    bot = _mm2(h02, h12, W_lin3[:D], W_lin3[D:], b_lin3)
    return jnp.concatenate([top, bot], axis=0)


def kernel(in_feat, W_lin, b_lin, W_lin2, b_lin2, W_sg1, b_sg1, W_sg2, b_sg2,
           W_lin3, b_lin3, W_lin4, b_lin4, W_c1, b_c1, W_c2, b_c2,
           sub_src0, sub_dst0, idx_nz0, idx_z0, lam0,
           sub_src1, sub_dst1, idx_nz1, idx_z1, lam1,
           sub_src2, sub_dst2, idx_nz2, idx_z2, lam2):
    h = _mm(in_feat, W_lin, b_lin, act="leaky")
    h = _mm(h, W_lin2, b_lin2, act="leaky")
    graphs = [(sub_src0, sub_dst0, idx_nz0, idx_z0, lam0),
              (sub_src1, sub_dst1, idx_nz1, idx_z1, lam1),
              (sub_src2, sub_dst2, idx_nz2, idx_z2, lam2)]
    acc = None
    for ss, sd, inz, iz, lam in graphs:
        h = _graph_stage(h, ss, sd, inz, iz, lam, W_c1, b_c1, W_c2, b_c2,
                         W_sg1, b_sg1, W_sg2, b_sg2, W_lin3, b_lin3)
        acc = h if acc is None else acc + h
    NCo = W_lin4.shape[1]
    W4p = jnp.zeros((W_lin4.shape[0], 128), jnp.float32).at[:, :NCo].set(W_lin4)
    b4p = jnp.zeros((128,), jnp.float32).at[:NCo].set(b_lin4)
    out = _mm(_act(acc, "leaky"), W4p, b4p)
    return out[:, :NCo]


# speedup vs baseline: 13.1996x; 13.1996x over previous
import functools

import jax
import jax.numpy as jnp
from jax import lax
from jax.experimental import pallas as pl
from jax.experimental.pallas import tpu as pltpu
import jax.experimental.pallas.tpu_sc as plsc

_NC = 2
_NS = 16
_EB = 128
_STG = 128


_BR = 1024


def _act(x, act):
    if act == "leaky":
        return jnp.where(x >= 0, x, 0.01 * x)
    if act == "relu":
        return jnp.maximum(x, 0.0)
    return x


def _mm1_body(a_ref, w_ref, b_ref, o_ref, *, act):
    x = jnp.dot(a_ref[...], w_ref[...], preferred_element_type=jnp.float32)
    o_ref[...] = _act(x + b_ref[...], act)


def _mm2_body(a_ref, b2_ref, w1_ref, w2_ref, b_ref, o_ref, *, act):
    x = jnp.dot(a_ref[...], w1_ref[...], preferred_element_type=jnp.float32)
    x = x + jnp.dot(b2_ref[...], w2_ref[...], preferred_element_type=jnp.float32)
    o_ref[...] = _act(x + b_ref[...], act)


def _mm(A, W, b, act=None):
    n, k = A.shape
    ko, m = W.shape
    grid = (pl.cdiv(n, _BR),)
    return pl.pallas_call(
        functools.partial(_mm1_body, act=act),
        grid=grid,
        in_specs=[
            pl.BlockSpec((_BR, k), lambda i: (i, 0)),
            pl.BlockSpec((ko, m), lambda i: (0, 0)),
            pl.BlockSpec((1, m), lambda i: (0, 0)),
        ],
        out_specs=pl.BlockSpec((_BR, m), lambda i: (i, 0)),
        out_shape=jax.ShapeDtypeStruct((n, m), jnp.float32),
    )(A, W, b.reshape(1, m))


def _mm2(A, B, W1, W2, b, act=None):
    n, k = A.shape
    m = W1.shape[1]
    grid = (pl.cdiv(n, _BR),)
    return pl.pallas_call(
        functools.partial(_mm2_body, act=act),
        grid=grid,
        in_specs=[
            pl.BlockSpec((_BR, k), lambda i: (i, 0)),
            pl.BlockSpec((_BR, B.shape[1]), lambda i: (i, 0)),
            pl.BlockSpec((W1.shape[0], m), lambda i: (0, 0)),
            pl.BlockSpec((W2.shape[0], m), lambda i: (0, 0)),
            pl.BlockSpec((1, m), lambda i: (0, 0)),
        ],
        out_specs=pl.BlockSpec((_BR, m), lambda i: (i, 0)),
        out_shape=jax.ShapeDtypeStruct((n, m), jnp.float32),
    )(A, B, W1, W2, b.reshape(1, m))


def _chunks(total):
    out = []
    off = 0
    while off < total:
        size = _STG if total - off >= _STG else 8
        out.append((off, size))
        off += size
    return out


def _zero_stage_2d(stg_v, rows, width):
    def zrow(j, _):
        for kk in range(width // 16):
            stg_v[j, pl.ds(kk * 16, 16)] = jnp.zeros((16,), jnp.float32)
        return ()

    lax.fori_loop(0, rows, zrow, ())


@functools.lru_cache(maxsize=None)
def _make_deg_kernel(n_ps, e_pads):
    mesh = plsc.VectorSubcoreMesh(
        core_axis_name="c", subcore_axis_name="s", num_cores=_NC,
        num_subcores=_NS)
    nbs = tuple(e // (_NS * _EB) for e in e_pads)
    scratch = [pltpu.VMEM((max(nbs), _EB), jnp.int32),
               pltpu.VMEM((_EB,), jnp.float32),
               pltpu.VMEM((512,), jnp.float32)]
    for n_p in n_ps:
        scratch.append(pltpu.VMEM_SHARED((n_p,), jnp.float32))
    out_type = tuple(jax.ShapeDtypeStruct((n_p,), jnp.float32) for n_p in n_ps)

    @functools.partial(pl.kernel, out_type=out_type, mesh=mesh,
                       scratch_types=scratch)
    def deg_kernel(sd0, sd1, sd2, o0, o1, o2,
                   idx_v, ones_v, stg_v, a0, a1, a2):
        c = lax.axis_index("c")
        s = lax.axis_index("s")
        for i in range(8):
            ones_v[pl.ds(i * 16, 16)] = jnp.full((16,), 1.0, jnp.float32)

        def do_graph(sd, out, acc, n_p, nb):
            rps = n_p // _NS
            base = s * rps

            def zs(j, _):
                stg_v[pl.ds(j * 16, 16)] = jnp.zeros((16,), jnp.float32)
                return ()

            lax.fori_loop(0, 32, zs, ())
            for off, size in _chunks(rps):
                pltpu.sync_copy(stg_v.at[pl.ds(0, min(size, 512))],
                                acc.at[pl.ds(base + off, min(size, 512))])
            pltpu.sync_copy(sd.at[pl.ds(s * nb, nb)], idx_v.at[pl.ds(0, nb)])
            plsc.subcore_barrier()

            def body(j, _):
                pltpu.sync_copy(ones_v, acc.at[idx_v.at[j]], add=True)
                return ()

            lax.fori_loop(0, nb, body, ())
            plsc.subcore_barrier()
            for off, size in _chunks(rps):
                sz = min(size, 512)
                pltpu.sync_copy(acc.at[pl.ds(base + off, sz)],
                                stg_v.at[pl.ds(0, sz)])
                pltpu.sync_copy(stg_v.at[pl.ds(0, sz)],
                                out.at[pl.ds(base + off, sz)])

        @pl.when(c == 0)
        def _():
            do_graph(sd0, o0, a0, n_ps[0], nbs[0])
            do_graph(sd1, o1, a1, n_ps[1], nbs[1])

        @pl.when(c == 1)
        def _():
            do_graph(sd2, o2, a2, n_ps[2], nbs[2])

    return deg_kernel


@functools.lru_cache(maxsize=None)
def _make_spmm_kernel(n_p, e_pad, width):
    nblk = e_pad // (_NC * _NS * _EB)
    rps = n_p // _NS
    mesh = plsc.VectorSubcoreMesh(
        core_axis_name="c", subcore_axis_name="s", num_cores=_NC,
        num_subcores=_NS)

    @functools.partial(
        pl.kernel,
        out_type=jax.ShapeDtypeStruct((_NC, n_p, width), jnp.float32),
        mesh=mesh,
        scratch_types=[
            pltpu.VMEM((nblk, _EB), jnp.int32),
            pltpu.VMEM((nblk, _EB), jnp.int32),
            pltpu.VMEM((_EB, width), jnp.float32),
            pltpu.VMEM((_STG, width), jnp.float32),
            pltpu.VMEM_SHARED((n_p, width), jnp.float32),
            pltpu.SemaphoreType.DMA,
        ])
    def spmm_kernel(xs, ss2d, sd2d, out, src_v, dst_v, rows_v, stg_v, acc, sem):
        c = lax.axis_index("c")
        s = lax.axis_index("s")
        w = c * _NS + s
        base = s * rps
        _zero_stage_2d(stg_v, _STG, width)
        pltpu.sync_copy(ss2d.at[pl.ds(w * nblk, nblk)], src_v)
        pltpu.sync_copy(sd2d.at[pl.ds(w * nblk, nblk)], dst_v)
        for off, size in _chunks(rps):
            pltpu.sync_copy(stg_v.at[pl.ds(0, size)],
                            acc.at[pl.ds(base + off, size)])
        plsc.subcore_barrier()

        def body(j, _):
            pltpu.async_copy(xs.at[src_v.at[j]], rows_v, sem).wait()
            pltpu.sync_copy(rows_v, acc.at[dst_v.at[j]], add=True)
            return ()

        lax.fori_loop(0, nblk, body, ())
        plsc.subcore_barrier()
        for off, size in _chunks(rps):
            pltpu.sync_copy(acc.at[pl.ds(base + off, size)],
                            stg_v.at[pl.ds(0, size)])
            pltpu.sync_copy(stg_v.at[pl.ds(0, size)],
                            out.at[c, pl.ds(base + off, size)])

    return spmm_kernel


def _pad_graph(ss, sd, n):
    e = ss.shape[0]
    n_p = (n // 128 + 2) * 128
    e_pad = -(-e // (8 * _NC * _NS * _EB)) * (8 * _NC * _NS * _EB)
    pad = e_pad - e
    scratch_rows = n_p - n
    pad_idx = n + (jnp.arange(pad, dtype=jnp.int32) % scratch_rows)
    ss_p = jnp.concatenate([ss, pad_idx])
    sd_p = jnp.concatenate([sd, pad_idx])
    return n_p, e_pad, ss_p, sd_p


def _graph_stage(h, inz, iz, lam, deg, n_p, e_pad, ss_p, sd_p,
                 W_c1, b_c1, W_c2, b_c2, W_sg1, b_sg1, W_sg2, b_sg2,
                 W_lin3, b_lin3):
    n = inz.shape[0]
    D = h.shape[1]
    dinv = jnp.clip(deg[:n], 1.0, None) ** -0.5
    rdinv = (2.0 / lam) * dinv
    h_nz = h[inz]
    h_z = h[iz]

    sd2d = sd_p.reshape(-1, _EB)
    ss2d = ss_p.reshape(-1, _EB)
    spmm = _make_spmm_kernel(n_p, e_pad, D)

    def cheb(X0, W, b):
        xs = jnp.pad(X0 * dinv[:, None], ((0, n_p - n), (0, 0)))
        msg2 = spmm(xs, ss2d, sd2d)
        X1s = rdinv[:, None] * (msg2[0, :n] + msg2[1, :n])
        return _mm2(X0, X1s, W[:D] - W[D:], W[D:], b, act="relu")

    h01 = cheb(h_nz, W_c1, b_c1)
    h11 = cheb(h01, W_c2, b_c2)
    top = _mm2(h01, h11, W_lin3[:D], W_lin3[D:], b_lin3)
    h02 = _mm(h_z, W_sg1, b_sg1)
    h12 = _mm(h02, W_sg2, b_sg2)
    bot = _mm2(h02, h12, W_lin3[:D], W_lin3[D:], b_lin3)
    return jnp.concatenate([top, bot], axis=0)


def kernel(in_feat, W_lin, b_lin, W_lin2, b_lin2, W_sg1, b_sg1, W_sg2, b_sg2,
           W_lin3, b_lin3, W_lin4, b_lin4, W_c1, b_c1, W_c2, b_c2,
           sub_src0, sub_dst0, idx_nz0, idx_z0, lam0,
           sub_src1, sub_dst1, idx_nz1, idx_z1, lam1,
           sub_src2, sub_dst2, idx_nz2, idx_z2, lam2):
    graphs = [(sub_src0, sub_dst0, idx_nz0, idx_z0, lam0),
              (sub_src1, sub_dst1, idx_nz1, idx_z1, lam1),
              (sub_src2, sub_dst2, idx_nz2, idx_z2, lam2)]

    pads = [_pad_graph(ss, sd, inz.shape[0])
            for ss, sd, inz, iz, lam in graphs]
    n_ps = tuple(p[0] for p in pads)
    e_pads = tuple(p[1] for p in pads)

    degk = _make_deg_kernel(n_ps, e_pads)
    degs = degk(pads[0][3].reshape(-1, _EB), pads[1][3].reshape(-1, _EB),
                pads[2][3].reshape(-1, _EB))

    h = _mm(in_feat, W_lin, b_lin, act="leaky")
    h = _mm(h, W_lin2, b_lin2, act="leaky")

    acc = None
    for g, (ss, sd, inz, iz, lam) in enumerate(graphs):
        n_p, e_pad, ss_p, sd_p = pads[g]
        h = _graph_stage(h, inz, iz, lam, degs[g], n_p, e_pad, ss_p, sd_p,
                         W_c1, b_c1, W_c2, b_c2, W_sg1, b_sg1, W_sg2, b_sg2,
                         W_lin3, b_lin3)
        acc = h if acc is None else acc + h

    NCo = W_lin4.shape[1]
    W4p = jnp.zeros((W_lin4.shape[0], 128), jnp.float32).at[:, :NCo].set(W_lin4)
    b4p = jnp.zeros((128,), jnp.float32).at[:NCo].set(b_lin4)
    out = _mm(_act(acc, "leaky"), W4p, b4p)
    return out[:, :NCo]


# speedup vs baseline: 18.4282x; 1.3961x over previous
import functools

import jax
import jax.numpy as jnp
from jax import lax
from jax.experimental import pallas as pl
from jax.experimental.pallas import tpu as pltpu
import jax.experimental.pallas.tpu_sc as plsc

_NC = 2
_NS = 16
_EB = 128
_STG = 128


_BR = 1024


def _act(x, act):
    if act == "leaky":
        return jnp.where(x >= 0, x, 0.01 * x)
    if act == "relu":
        return jnp.maximum(x, 0.0)
    return x


def _mm1_body(a_ref, w_ref, b_ref, o_ref, *, act):
    x = jnp.dot(a_ref[...], w_ref[...], preferred_element_type=jnp.float32)
    o_ref[...] = _act(x + b_ref[...], act)


def _mm2_body(a_ref, b2_ref, w1_ref, w2_ref, b_ref, o_ref, *, act):
    x = jnp.dot(a_ref[...], w1_ref[...], preferred_element_type=jnp.float32)
    x = x + jnp.dot(b2_ref[...], w2_ref[...], preferred_element_type=jnp.float32)
    o_ref[...] = _act(x + b_ref[...], act)


def _mm(A, W, b, act=None):
    n, k = A.shape
    ko, m = W.shape
    grid = (pl.cdiv(n, _BR),)
    return pl.pallas_call(
        functools.partial(_mm1_body, act=act),
        grid=grid,
        in_specs=[
            pl.BlockSpec((_BR, k), lambda i: (i, 0)),
            pl.BlockSpec((ko, m), lambda i: (0, 0)),
            pl.BlockSpec((1, m), lambda i: (0, 0)),
        ],
        out_specs=pl.BlockSpec((_BR, m), lambda i: (i, 0)),
        out_shape=jax.ShapeDtypeStruct((n, m), jnp.float32),
    )(A, W, b.reshape(1, m))


def _mm3_body(a_ref, b2_ref, c3_ref, w1_ref, w2_ref, w3_ref, b_ref, o_ref, *, act):
    x = jnp.dot(a_ref[...], w1_ref[...], preferred_element_type=jnp.float32)
    x = x + jnp.dot(b2_ref[...], w2_ref[...], preferred_element_type=jnp.float32)
    x = x + jnp.dot(c3_ref[...], w3_ref[...], preferred_element_type=jnp.float32)
    o_ref[...] = _act(x + b_ref[...], act)


def _mm3(A, B, C, W1, W2, W3, b, act=None):
    n, k = A.shape
    m = W1.shape[1]
    grid = (pl.cdiv(n, _BR),)
    return pl.pallas_call(
        functools.partial(_mm3_body, act=act),
        grid=grid,
        in_specs=[
            pl.BlockSpec((_BR, k), lambda i: (i, 0)),
            pl.BlockSpec((_BR, B.shape[1]), lambda i: (i, 0)),
            pl.BlockSpec((_BR, C.shape[1]), lambda i: (i, 0)),
            pl.BlockSpec((W1.shape[0], m), lambda i: (0, 0)),
            pl.BlockSpec((W2.shape[0], m), lambda i: (0, 0)),
            pl.BlockSpec((W3.shape[0], m), lambda i: (0, 0)),
            pl.BlockSpec((1, m), lambda i: (0, 0)),
        ],
        out_specs=pl.BlockSpec((_BR, m), lambda i: (i, 0)),
        out_shape=jax.ShapeDtypeStruct((n, m), jnp.float32),
    )(A, B, C, W1, W2, W3, b.reshape(1, m))


def _mm2(A, B, W1, W2, b, act=None):
    n, k = A.shape
    m = W1.shape[1]
    grid = (pl.cdiv(n, _BR),)
    return pl.pallas_call(
        functools.partial(_mm2_body, act=act),
        grid=grid,
        in_specs=[
            pl.BlockSpec((_BR, k), lambda i: (i, 0)),
            pl.BlockSpec((_BR, B.shape[1]), lambda i: (i, 0)),
            pl.BlockSpec((W1.shape[0], m), lambda i: (0, 0)),
            pl.BlockSpec((W2.shape[0], m), lambda i: (0, 0)),
            pl.BlockSpec((1, m), lambda i: (0, 0)),
        ],
        out_specs=pl.BlockSpec((_BR, m), lambda i: (i, 0)),
        out_shape=jax.ShapeDtypeStruct((n, m), jnp.float32),
    )(A, B, W1, W2, b.reshape(1, m))


def _chunks(total):
    out = []
    off = 0
    while off < total:
        size = _STG if total - off >= _STG else 8
        out.append((off, size))
        off += size
    return out


def _zero_stage_2d(stg_v, rows, width):
    def zrow(j, _):
        for kk in range(width // 16):
            stg_v[j, pl.ds(kk * 16, 16)] = jnp.zeros((16,), jnp.float32)
        return ()

    lax.fori_loop(0, rows, zrow, ())


@functools.lru_cache(maxsize=None)
def _make_deg_kernel(n_ps, e_pads):
    mesh = plsc.VectorSubcoreMesh(
        core_axis_name="c", subcore_axis_name="s", num_cores=_NC,
        num_subcores=_NS)
    nbs = tuple(e // (_NS * _EB) for e in e_pads)
    scratch = [pltpu.VMEM((max(nbs), _EB), jnp.int32),
               pltpu.VMEM((_EB,), jnp.float32),
               pltpu.VMEM((512,), jnp.float32)]
    for n_p in n_ps:
        scratch.append(pltpu.VMEM_SHARED((n_p,), jnp.float32))
    out_type = tuple(jax.ShapeDtypeStruct((n_p,), jnp.float32) for n_p in n_ps)

    @functools.partial(pl.kernel, out_type=out_type, mesh=mesh,
                       scratch_types=scratch)
    def deg_kernel(sd0, sd1, sd2, o0, o1, o2,
                   idx_v, ones_v, stg_v, a0, a1, a2):
        c = lax.axis_index("c")
        s = lax.axis_index("s")
        for i in range(8):
            ones_v[pl.ds(i * 16, 16)] = jnp.full((16,), 1.0, jnp.float32)

        def do_graph(sd, out, acc, n_p, nb):
            rps = n_p // _NS
            base = s * rps

            def zs(j, _):
                stg_v[pl.ds(j * 16, 16)] = jnp.zeros((16,), jnp.float32)
                return ()

            lax.fori_loop(0, 32, zs, ())
            for off, size in _chunks(rps):
                pltpu.sync_copy(stg_v.at[pl.ds(0, min(size, 512))],
                                acc.at[pl.ds(base + off, min(size, 512))])
            pltpu.sync_copy(sd.at[pl.ds(s * nb, nb)], idx_v.at[pl.ds(0, nb)])
            plsc.subcore_barrier()

            def body(j, _):
                pltpu.sync_copy(ones_v, acc.at[idx_v.at[j]], add=True)
                return ()

            lax.fori_loop(0, nb, body, ())
            plsc.subcore_barrier()
            for off, size in _chunks(rps):
                sz = min(size, 512)
                pltpu.sync_copy(acc.at[pl.ds(base + off, sz)],
                                stg_v.at[pl.ds(0, sz)])
                pltpu.sync_copy(stg_v.at[pl.ds(0, sz)],
                                out.at[pl.ds(base + off, sz)])

        @pl.when(c == 0)
        def _():
            do_graph(sd0, o0, a0, n_ps[0], nbs[0])
            do_graph(sd1, o1, a1, n_ps[1], nbs[1])

        @pl.when(c == 1)
        def _():
            do_graph(sd2, o2, a2, n_ps[2], nbs[2])

    return deg_kernel


@functools.lru_cache(maxsize=None)
def _make_spmm_kernel(n_p, e_pad, width):
    nblk = e_pad // (_NC * _NS * _EB)
    rps = n_p // _NS
    mesh = plsc.VectorSubcoreMesh(
        core_axis_name="c", subcore_axis_name="s", num_cores=_NC,
        num_subcores=_NS)

    @functools.partial(
        pl.kernel,
        out_type=jax.ShapeDtypeStruct((_NC, n_p, width), jnp.float32),
        mesh=mesh,
        scratch_types=[
            pltpu.VMEM((nblk, _EB), jnp.int32),
            pltpu.VMEM((nblk, _EB), jnp.int32),
            pltpu.VMEM((_EB, width), jnp.float32),
            pltpu.VMEM((_EB, width), jnp.float32),
            pltpu.VMEM_SHARED((n_p, width), jnp.float32),
            pltpu.SemaphoreType.DMA,
            pltpu.SemaphoreType.DMA,
        ])
    def spmm_kernel(xs, ss2d, sd2d, out, src_v, dst_v, rows_a, rows_b,
                    acc, sem_a, sem_b):
        c = lax.axis_index("c")
        s = lax.axis_index("s")
        w = c * _NS + s
        base = s * rps
        _zero_stage_2d(rows_a, _STG, width)
        pltpu.sync_copy(ss2d.at[pl.ds(w * nblk, nblk)], src_v)
        pltpu.sync_copy(sd2d.at[pl.ds(w * nblk, nblk)], dst_v)
        for off, size in _chunks(rps):
            pltpu.sync_copy(rows_a.at[pl.ds(0, size)],
                            acc.at[pl.ds(base + off, size)])
        plsc.subcore_barrier()

        def gath(j, buf, sem):
            pltpu.async_copy(xs.at[src_v.at[j]], buf, sem)

        def gwait(j, buf, sem):
            pltpu.make_async_copy(xs.at[src_v.at[j]], buf, sem).wait()

        def scat(j, buf):
            pltpu.sync_copy(buf, acc.at[dst_v.at[j]], add=True)

        nblk2 = nblk // 2

        def body(i, _):
            ja, jb = 2 * i, 2 * i + 1

            @pl.when(i == 0)
            def _():
                gath(ja, rows_a, sem_a)

            gath(jb, rows_b, sem_b)
            gwait(ja, rows_a, sem_a)
            scat(ja, rows_a)

            @pl.when(i < nblk2 - 1)
            def _():
                gath(jb + 1, rows_a, sem_a)

            gwait(jb, rows_b, sem_b)
            scat(jb, rows_b)
            return ()

        lax.fori_loop(0, nblk2, body, ())
        plsc.subcore_barrier()
        for off, size in _chunks(rps):
            pltpu.sync_copy(acc.at[pl.ds(base + off, size)],
                            rows_a.at[pl.ds(0, size)])
            pltpu.sync_copy(rows_a.at[pl.ds(0, size)],
                            out.at[c, pl.ds(base + off, size)])

    return spmm_kernel


def _pad_graph(ss, sd, n):
    e = ss.shape[0]
    n_p = (n // 128 + 2) * 128
    e_pad = -(-e // (8 * _NC * _NS * _EB)) * (8 * _NC * _NS * _EB)
    pad = e_pad - e
    scratch_rows = n_p - n
    pad_idx = n + (jnp.arange(pad, dtype=jnp.int32) % scratch_rows)
    ss_p = jnp.concatenate([ss, pad_idx])
    sd_p = jnp.concatenate([sd, pad_idx])
    return n_p, e_pad, ss_p, sd_p


def _graph_stage(h, inz, iz, lam, deg, n_p, e_pad, ss_p, sd_p,
                 W_c1, b_c1, W_c2, b_c2, W_sg1, b_sg1, W_sg2, b_sg2,
                 W_lin3, b_lin3):
    n = inz.shape[0]
    D = h.shape[1]
    dinv = jnp.clip(deg[:n], 1.0, None) ** -0.5
    rdinv = (2.0 / lam) * dinv
    h_nz = h[inz]
    h_z = h[iz]

    sd2d = sd_p.reshape(-1, _EB)
    ss2d = ss_p.reshape(-1, _EB)
    spmm = _make_spmm_kernel(n_p, e_pad, D)

    def cheb(X0, W, b):
        xs = jnp.pad(X0 * dinv[:, None], ((0, n_p - n), (0, 0)))
        msg2 = spmm(xs, ss2d, sd2d)
        BL = rdinv[:, None] * msg2[0, :n]
        BR = rdinv[:, None] * msg2[1, :n]
        Wb = W[D:]
        return _mm3(X0, BL, BR, W[:D] - Wb, Wb, Wb, b, act="relu")

    h01 = cheb(h_nz, W_c1, b_c1)
    h11 = cheb(h01, W_c2, b_c2)
    top = _mm2(h01, h11, W_lin3[:D], W_lin3[D:], b_lin3)
    h02 = _mm(h_z, W_sg1, b_sg1)
    h12 = _mm(h02, W_sg2, b_sg2)
    bot = _mm2(h02, h12, W_lin3[:D], W_lin3[D:], b_lin3)
    return jnp.concatenate([top, bot], axis=0)


def kernel(in_feat, W_lin, b_lin, W_lin2, b_lin2, W_sg1, b_sg1, W_sg2, b_sg2,
           W_lin3, b_lin3, W_lin4, b_lin4, W_c1, b_c1, W_c2, b_c2,
           sub_src0, sub_dst0, idx_nz0, idx_z0, lam0,
           sub_src1, sub_dst1, idx_nz1, idx_z1, lam1,
           sub_src2, sub_dst2, idx_nz2, idx_z2, lam2):
    graphs = [(sub_src0, sub_dst0, idx_nz0, idx_z0, lam0),
              (sub_src1, sub_dst1, idx_nz1, idx_z1, lam1),
              (sub_src2, sub_dst2, idx_nz2, idx_z2, lam2)]

    pads = [_pad_graph(ss, sd, inz.shape[0])
            for ss, sd, inz, iz, lam in graphs]
    n_ps = tuple(p[0] for p in pads)
    e_pads = tuple(p[1] for p in pads)

    degk = _make_deg_kernel(n_ps, e_pads)
    degs = degk(pads[0][3].reshape(-1, _EB), pads[1][3].reshape(-1, _EB),
                pads[2][3].reshape(-1, _EB))

    h = _mm(in_feat, W_lin, b_lin, act="leaky")
    h = _mm(h, W_lin2, b_lin2, act="leaky")

    acc = None
    for g, (ss, sd, inz, iz, lam) in enumerate(graphs):
        n_p, e_pad, ss_p, sd_p = pads[g]
        h = _graph_stage(h, inz, iz, lam, degs[g], n_p, e_pad, ss_p, sd_p,
                         W_c1, b_c1, W_c2, b_c2, W_sg1, b_sg1, W_sg2, b_sg2,
                         W_lin3, b_lin3)
        acc = h if acc is None else acc + h

    NCo = W_lin4.shape[1]
    W4p = jnp.zeros((W_lin4.shape[0], 128), jnp.float32).at[:, :NCo].set(W_lin4)
    b4p = jnp.zeros((128,), jnp.float32).at[:NCo].set(b_lin4)
    out = _mm(_act(acc, "leaky"), W4p, b4p)
    return out[:, :NCo]


# speedup vs baseline: 18.4476x; 1.0011x over previous
import functools

import jax
import jax.numpy as jnp
from jax import lax
from jax.experimental import pallas as pl
from jax.experimental.pallas import tpu as pltpu
import jax.experimental.pallas.tpu_sc as plsc

_NC = 2
_NS = 16
_EB = 128
_STG = 128


_BR = 1024


def _act(x, act):
    if act == "leaky":
        return jnp.where(x >= 0, x, 0.01 * x)
    if act == "relu":
        return jnp.maximum(x, 0.0)
    return x


def _mm1_body(a_ref, w_ref, b_ref, o_ref, *, act):
    x = jnp.dot(a_ref[...], w_ref[...], preferred_element_type=jnp.float32)
    o_ref[...] = _act(x + b_ref[...], act)


def _mm2_body(a_ref, b2_ref, w1_ref, w2_ref, b_ref, o_ref, *, act):
    x = jnp.dot(a_ref[...], w1_ref[...], preferred_element_type=jnp.float32)
    x = x + jnp.dot(b2_ref[...], w2_ref[...], preferred_element_type=jnp.float32)
    o_ref[...] = _act(x + b_ref[...], act)


def _mm(A, W, b, act=None):
    n, k = A.shape
    ko, m = W.shape
    grid = (pl.cdiv(n, _BR),)
    return pl.pallas_call(
        functools.partial(_mm1_body, act=act),
        grid=grid,
        in_specs=[
            pl.BlockSpec((_BR, k), lambda i: (i, 0)),
            pl.BlockSpec((ko, m), lambda i: (0, 0)),
            pl.BlockSpec((1, m), lambda i: (0, 0)),
        ],
        out_specs=pl.BlockSpec((_BR, m), lambda i: (i, 0)),
        out_shape=jax.ShapeDtypeStruct((n, m), jnp.float32),
    )(A, W, b.reshape(1, m))


def _mm3_body(a_ref, b2_ref, c3_ref, w1_ref, w2_ref, w3_ref, b_ref, o_ref, *, act):
    x = jnp.dot(a_ref[...], w1_ref[...], preferred_element_type=jnp.float32)
    x = x + jnp.dot(b2_ref[...], w2_ref[...], preferred_element_type=jnp.float32)
    x = x + jnp.dot(c3_ref[...], w3_ref[...], preferred_element_type=jnp.float32)
    o_ref[...] = _act(x + b_ref[...], act)


def _mm3(A, B, C, W1, W2, W3, b, act=None):
    n, k = A.shape
    m = W1.shape[1]
    grid = (pl.cdiv(n, _BR),)
    return pl.pallas_call(
        functools.partial(_mm3_body, act=act),
        grid=grid,
        in_specs=[
            pl.BlockSpec((_BR, k), lambda i: (i, 0)),
            pl.BlockSpec((_BR, B.shape[1]), lambda i: (i, 0)),
            pl.BlockSpec((_BR, C.shape[1]), lambda i: (i, 0)),
            pl.BlockSpec((W1.shape[0], m), lambda i: (0, 0)),
            pl.BlockSpec((W2.shape[0], m), lambda i: (0, 0)),
            pl.BlockSpec((W3.shape[0], m), lambda i: (0, 0)),
            pl.BlockSpec((1, m), lambda i: (0, 0)),
        ],
        out_specs=pl.BlockSpec((_BR, m), lambda i: (i, 0)),
        out_shape=jax.ShapeDtypeStruct((n, m), jnp.float32),
    )(A, B, C, W1, W2, W3, b.reshape(1, m))


def _zbranch_body(a_ref, ws1_ref, bs1_ref, ws2_ref, bs2_ref, w3a_ref,
                  w3b_ref, b3_ref, o_ref):
    h02 = jnp.dot(a_ref[...], ws1_ref[...],
                  preferred_element_type=jnp.float32) + bs1_ref[...]
    h12 = jnp.dot(h02, ws2_ref[...],
                  preferred_element_type=jnp.float32) + bs2_ref[...]
    x = jnp.dot(h02, w3a_ref[...], preferred_element_type=jnp.float32)
    x = x + jnp.dot(h12, w3b_ref[...], preferred_element_type=jnp.float32)
    o_ref[...] = x + b3_ref[...]


def _zbranch(hz, Ws1, bs1, Ws2, bs2, W3a, W3b, b3):
    n, k = hz.shape
    m = W3a.shape[1]
    grid = (pl.cdiv(n, _BR),)
    full = lambda r, c: pl.BlockSpec((r, c), lambda i: (0, 0))
    return pl.pallas_call(
        _zbranch_body,
        grid=grid,
        in_specs=[
            pl.BlockSpec((_BR, k), lambda i: (i, 0)),
            full(k, m), full(1, m), full(m, m), full(1, m),
            full(m, m), full(m, m), full(1, m),
        ],
        out_specs=pl.BlockSpec((_BR, m), lambda i: (i, 0)),
        out_shape=jax.ShapeDtypeStruct((n, m), jnp.float32),
    )(hz, Ws1, bs1.reshape(1, -1), Ws2, bs2.reshape(1, -1),
      W3a, W3b, b3.reshape(1, -1))


def _final3_body(a_ref, b_ref, c_ref, w_ref, bias_ref, o_ref):
    x = a_ref[...] + b_ref[...] + c_ref[...]
    x = jnp.where(x >= 0, x, 0.01 * x)
    o_ref[...] = jnp.dot(x, w_ref[...],
                         preferred_element_type=jnp.float32) + bias_ref[...]


def _final3(h1, h2, h3, W, b):
    n, k = h1.shape
    m = W.shape[1]
    grid = (pl.cdiv(n, _BR),)
    return pl.pallas_call(
        _final3_body,
        grid=grid,
        in_specs=[
            pl.BlockSpec((_BR, k), lambda i: (i, 0)),
            pl.BlockSpec((_BR, k), lambda i: (i, 0)),
            pl.BlockSpec((_BR, k), lambda i: (i, 0)),
            pl.BlockSpec((k, m), lambda i: (0, 0)),
            pl.BlockSpec((1, m), lambda i: (0, 0)),
        ],
        out_specs=pl.BlockSpec((_BR, m), lambda i: (i, 0)),
        out_shape=jax.ShapeDtypeStruct((n, m), jnp.float32),
    )(h1, h2, h3, W, b.reshape(1, m))


def _mm2(A, B, W1, W2, b, act=None):
    n, k = A.shape
    m = W1.shape[1]
    grid = (pl.cdiv(n, _BR),)
    return pl.pallas_call(
        functools.partial(_mm2_body, act=act),
        grid=grid,
        in_specs=[
            pl.BlockSpec((_BR, k), lambda i: (i, 0)),
            pl.BlockSpec((_BR, B.shape[1]), lambda i: (i, 0)),
            pl.BlockSpec((W1.shape[0], m), lambda i: (0, 0)),
            pl.BlockSpec((W2.shape[0], m), lambda i: (0, 0)),
            pl.BlockSpec((1, m), lambda i: (0, 0)),
        ],
        out_specs=pl.BlockSpec((_BR, m), lambda i: (i, 0)),
        out_shape=jax.ShapeDtypeStruct((n, m), jnp.float32),
    )(A, B, W1, W2, b.reshape(1, m))


def _chunks(total):
    out = []
    off = 0
    while off < total:
        size = _STG if total - off >= _STG else 8
        out.append((off, size))
        off += size
    return out


def _zero_stage_2d(stg_v, rows, width):
    def zrow(j, _):
        for kk in range(width // 16):
            stg_v[j, pl.ds(kk * 16, 16)] = jnp.zeros((16,), jnp.float32)
        return ()

    lax.fori_loop(0, rows, zrow, ())


@functools.lru_cache(maxsize=None)
def _make_deg_kernel(n_ps, e_pads):
    mesh = plsc.VectorSubcoreMesh(
        core_axis_name="c", subcore_axis_name="s", num_cores=_NC,
        num_subcores=_NS)
    nbs = tuple(e // (_NS * _EB) for e in e_pads)
    scratch = [pltpu.VMEM((max(nbs), _EB), jnp.int32),
               pltpu.VMEM((_EB,), jnp.float32),
               pltpu.VMEM((512,), jnp.float32)]
    for n_p in n_ps:
        scratch.append(pltpu.VMEM_SHARED((n_p,), jnp.float32))
    out_type = tuple(jax.ShapeDtypeStruct((n_p,), jnp.float32) for n_p in n_ps)

    @functools.partial(pl.kernel, out_type=out_type, mesh=mesh,
                       scratch_types=scratch)
    def deg_kernel(sd0, sd1, sd2, o0, o1, o2,
                   idx_v, ones_v, stg_v, a0, a1, a2):
        c = lax.axis_index("c")
        s = lax.axis_index("s")
        for i in range(8):
            ones_v[pl.ds(i * 16, 16)] = jnp.full((16,), 1.0, jnp.float32)

        def do_graph(sd, out, acc, n_p, nb):
            rps = n_p // _NS
            base = s * rps

            def zs(j, _):
                stg_v[pl.ds(j * 16, 16)] = jnp.zeros((16,), jnp.float32)
                return ()

            lax.fori_loop(0, 32, zs, ())
            for off, size in _chunks(rps):
                pltpu.sync_copy(stg_v.at[pl.ds(0, min(size, 512))],
                                acc.at[pl.ds(base + off, min(size, 512))])
            pltpu.sync_copy(sd.at[pl.ds(s * nb, nb)], idx_v.at[pl.ds(0, nb)])
            plsc.subcore_barrier()

            def body(j, _):
                pltpu.sync_copy(ones_v, acc.at[idx_v.at[j]], add=True)
                return ()

            lax.fori_loop(0, nb, body, ())
            plsc.subcore_barrier()
            for off, size in _chunks(rps):
                sz = min(size, 512)
                pltpu.sync_copy(acc.at[pl.ds(base + off, sz)],
                                stg_v.at[pl.ds(0, sz)])
                pltpu.sync_copy(stg_v.at[pl.ds(0, sz)],
                                out.at[pl.ds(base + off, sz)])

        @pl.when(c == 0)
        def _():
            do_graph(sd0, o0, a0, n_ps[0], nbs[0])
            do_graph(sd1, o1, a1, n_ps[1], nbs[1])

        @pl.when(c == 1)
        def _():
            do_graph(sd2, o2, a2, n_ps[2], nbs[2])

    return deg_kernel


@functools.lru_cache(maxsize=None)
def _make_spmm_kernel(n_p, e_pad, width):
    nblk = e_pad // (_NC * _NS * _EB)
    rps = n_p // _NS
    mesh = plsc.VectorSubcoreMesh(
        core_axis_name="c", subcore_axis_name="s", num_cores=_NC,
        num_subcores=_NS)

    @functools.partial(
        pl.kernel,
        out_type=jax.ShapeDtypeStruct((_NC, n_p, width), jnp.float32),
        mesh=mesh,
        scratch_types=[
            pltpu.VMEM((nblk, _EB), jnp.int32),
            pltpu.VMEM((nblk, _EB), jnp.int32),
            pltpu.VMEM((_EB, width), jnp.float32),
            pltpu.VMEM((_EB, width), jnp.float32),
            pltpu.VMEM_SHARED((n_p, width), jnp.float32),
            pltpu.SemaphoreType.DMA,
            pltpu.SemaphoreType.DMA,
        ])
    def spmm_kernel(xs, ss2d, sd2d, out, src_v, dst_v, rows_a, rows_b,
                    acc, sem_a, sem_b):
        c = lax.axis_index("c")
        s = lax.axis_index("s")
        w = c * _NS + s
        base = s * rps
        _zero_stage_2d(rows_a, _STG, width)
        pltpu.sync_copy(ss2d.at[pl.ds(w * nblk, nblk)], src_v)
        pltpu.sync_copy(sd2d.at[pl.ds(w * nblk, nblk)], dst_v)
        for off, size in _chunks(rps):
            pltpu.sync_copy(rows_a.at[pl.ds(0, size)],
                            acc.at[pl.ds(base + off, size)])
        plsc.subcore_barrier()

        def gath(j, buf, sem):
            pltpu.async_copy(xs.at[src_v.at[j]], buf, sem)

        def gwait(j, buf, sem):
            pltpu.make_async_copy(xs.at[src_v.at[j]], buf, sem).wait()

        def scat(j, buf):
            pltpu.sync_copy(buf, acc.at[dst_v.at[j]], add=True)

        nblk2 = nblk // 2

        def body(i, _):
            ja, jb = 2 * i, 2 * i + 1

            @pl.when(i == 0)
            def _():
                gath(ja, rows_a, sem_a)

            gath(jb, rows_b, sem_b)
            gwait(ja, rows_a, sem_a)
            scat(ja, rows_a)

            @pl.when(i < nblk2 - 1)
            def _():
                gath(jb + 1, rows_a, sem_a)

            gwait(jb, rows_b, sem_b)
            scat(jb, rows_b)
            return ()

        lax.fori_loop(0, nblk2, body, ())
        plsc.subcore_barrier()
        for off, size in _chunks(rps):
            pltpu.sync_copy(acc.at[pl.ds(base + off, size)],
                            rows_a.at[pl.ds(0, size)])
            pltpu.sync_copy(rows_a.at[pl.ds(0, size)],
                            out.at[c, pl.ds(base + off, size)])

    return spmm_kernel


def _pad_graph(ss, sd, n):
    e = ss.shape[0]
    n_p = (n // 128 + 2) * 128
    e_pad = -(-e // (8 * _NC * _NS * _EB)) * (8 * _NC * _NS * _EB)
    pad = e_pad - e
    scratch_rows = n_p - n
    pad_idx = n + (jnp.arange(pad, dtype=jnp.int32) % scratch_rows)
    ss_p = jnp.concatenate([ss, pad_idx])
    sd_p = jnp.concatenate([sd, pad_idx])
    return n_p, e_pad, ss_p, sd_p


def _graph_stage(h, inz, iz, lam, deg, n_p, e_pad, ss_p, sd_p,
                 W_c1, b_c1, W_c2, b_c2, W_sg1, b_sg1, W_sg2, b_sg2,
                 W_lin3, b_lin3):
    n = inz.shape[0]
    D = h.shape[1]
    dinv = jnp.clip(deg[:n], 1.0, None) ** -0.5
    rdinv = (2.0 / lam) * dinv
    h_nz = h[inz]
    h_z = h[iz]

    sd2d = sd_p.reshape(-1, _EB)
    ss2d = ss_p.reshape(-1, _EB)
    spmm = _make_spmm_kernel(n_p, e_pad, D)

    def cheb(X0, W, b):
        xs = jnp.pad(X0 * dinv[:, None], ((0, n_p - n), (0, 0)))
        msg2 = spmm(xs, ss2d, sd2d)
        BL = rdinv[:, None] * msg2[0, :n]
        BR = rdinv[:, None] * msg2[1, :n]
        Wb = W[D:]
        return _mm3(X0, BL, BR, W[:D] - Wb, Wb, Wb, b, act="relu")

    h01 = cheb(h_nz, W_c1, b_c1)
    h11 = cheb(h01, W_c2, b_c2)
    top = _mm2(h01, h11, W_lin3[:D], W_lin3[D:], b_lin3)
    bot = _zbranch(h_z, W_sg1, b_sg1, W_sg2, b_sg2,
                   W_lin3[:D], W_lin3[D:], b_lin3)
    return jnp.concatenate([top, bot], axis=0)


def kernel(in_feat, W_lin, b_lin, W_lin2, b_lin2, W_sg1, b_sg1, W_sg2, b_sg2,
           W_lin3, b_lin3, W_lin4, b_lin4, W_c1, b_c1, W_c2, b_c2,
           sub_src0, sub_dst0, idx_nz0, idx_z0, lam0,
           sub_src1, sub_dst1, idx_nz1, idx_z1, lam1,
           sub_src2, sub_dst2, idx_nz2, idx_z2, lam2):
    graphs = [(sub_src0, sub_dst0, idx_nz0, idx_z0, lam0),
              (sub_src1, sub_dst1, idx_nz1, idx_z1, lam1),
              (sub_src2, sub_dst2, idx_nz2, idx_z2, lam2)]

    pads = [_pad_graph(ss, sd, inz.shape[0])
            for ss, sd, inz, iz, lam in graphs]
    n_ps = tuple(p[0] for p in pads)
    e_pads = tuple(p[1] for p in pads)

    degk = _make_deg_kernel(n_ps, e_pads)
    degs = degk(pads[0][3].reshape(-1, _EB), pads[1][3].reshape(-1, _EB),
                pads[2][3].reshape(-1, _EB))

    h = _mm(in_feat, W_lin, b_lin, act="leaky")
    h = _mm(h, W_lin2, b_lin2, act="leaky")

    hs = []
    for g, (ss, sd, inz, iz, lam) in enumerate(graphs):
        n_p, e_pad, ss_p, sd_p = pads[g]
        h = _graph_stage(h, inz, iz, lam, degs[g], n_p, e_pad, ss_p, sd_p,
                         W_c1, b_c1, W_c2, b_c2, W_sg1, b_sg1, W_sg2, b_sg2,
                         W_lin3, b_lin3)
        hs.append(h)

    NCo = W_lin4.shape[1]
    W4p = jnp.zeros((W_lin4.shape[0], 128), jnp.float32).at[:, :NCo].set(W_lin4)
    b4p = jnp.zeros((128,), jnp.float32).at[:NCo].set(b_lin4)
    out = _final3(hs[0], hs[1], hs[2], W4p, b4p)
    return out[:, :NCo]


# speedup vs baseline: 18.6809x; 1.0126x over previous
import functools

import jax
import jax.numpy as jnp
from jax import lax
from jax.experimental import pallas as pl
from jax.experimental.pallas import tpu as pltpu
import jax.experimental.pallas.tpu_sc as plsc

_NC = 2
_NS = 16
_EB = 128
_STG = 128


_BR = 1024


def _act(x, act):
    if act == "leaky":
        return jnp.where(x >= 0, x, 0.01 * x)
    if act == "relu":
        return jnp.maximum(x, 0.0)
    return x


def _mm1_body(a_ref, w_ref, b_ref, o_ref, *, act):
    x = jnp.dot(a_ref[...], w_ref[...], preferred_element_type=jnp.float32)
    o_ref[...] = _act(x + b_ref[...], act)


def _mm2_body(a_ref, b2_ref, w1_ref, w2_ref, b_ref, o_ref, *, act):
    x = jnp.dot(a_ref[...], w1_ref[...], preferred_element_type=jnp.float32)
    x = x + jnp.dot(b2_ref[...], w2_ref[...], preferred_element_type=jnp.float32)
    o_ref[...] = _act(x + b_ref[...], act)


def _mm(A, W, b, act=None):
    n, k = A.shape
    ko, m = W.shape
    grid = (pl.cdiv(n, _BR),)
    return pl.pallas_call(
        functools.partial(_mm1_body, act=act),
        grid=grid,
        in_specs=[
            pl.BlockSpec((_BR, k), lambda i: (i, 0)),
            pl.BlockSpec((ko, m), lambda i: (0, 0)),
            pl.BlockSpec((1, m), lambda i: (0, 0)),
        ],
        out_specs=pl.BlockSpec((_BR, m), lambda i: (i, 0)),
        out_shape=jax.ShapeDtypeStruct((n, m), jnp.float32),
    )(A, W, b.reshape(1, m))


def _mm3_body(a_ref, b2_ref, c3_ref, w1_ref, w2_ref, w3_ref, b_ref, o_ref, *, act):
    x = jnp.dot(a_ref[...], w1_ref[...], preferred_element_type=jnp.float32)
    x = x + jnp.dot(b2_ref[...], w2_ref[...], preferred_element_type=jnp.float32)
    x = x + jnp.dot(c3_ref[...], w3_ref[...], preferred_element_type=jnp.float32)
    o_ref[...] = _act(x + b_ref[...], act)


def _mm3(A, B, C, W1, W2, W3, b, act=None):
    n, k = A.shape
    m = W1.shape[1]
    grid = (pl.cdiv(n, _BR),)
    return pl.pallas_call(
        functools.partial(_mm3_body, act=act),
        grid=grid,
        in_specs=[
            pl.BlockSpec((_BR, k), lambda i: (i, 0)),
            pl.BlockSpec((_BR, B.shape[1]), lambda i: (i, 0)),
            pl.BlockSpec((_BR, C.shape[1]), lambda i: (i, 0)),
            pl.BlockSpec((W1.shape[0], m), lambda i: (0, 0)),
            pl.BlockSpec((W2.shape[0], m), lambda i: (0, 0)),
            pl.BlockSpec((W3.shape[0], m), lambda i: (0, 0)),
            pl.BlockSpec((1, m), lambda i: (0, 0)),
        ],
        out_specs=pl.BlockSpec((_BR, m), lambda i: (i, 0)),
        out_shape=jax.ShapeDtypeStruct((n, m), jnp.float32),
    )(A, B, C, W1, W2, W3, b.reshape(1, m))


def _cheb_body(x0_ref, m0_ref, m1_ref, dv_ref, wd_ref, wbr_ref, b_ref,
               o_ref, xs_ref):
    x1s = dv_ref[...] * (m0_ref[...] + m1_ref[...])
    x = jnp.dot(x0_ref[...], wd_ref[...], preferred_element_type=jnp.float32)
    x = x + jnp.dot(x1s, wbr_ref[...], preferred_element_type=jnp.float32)
    o = jnp.maximum(x + b_ref[...], 0.0)
    o_ref[...] = o
    xs_ref[...] = dv_ref[...] * o


def _cheb_mm(X0, m0, m1, dinv_b, Wd, Wbr, b, n_p):
    n, k = X0.shape
    m = Wd.shape[1]
    grid = (pl.cdiv(n, _BR),)
    row = pl.BlockSpec((_BR, k), lambda i: (i, 0))
    return pl.pallas_call(
        _cheb_body,
        grid=grid,
        in_specs=[row, row, row, row,
                  pl.BlockSpec((k, m), lambda i: (0, 0)),
                  pl.BlockSpec((k, m), lambda i: (0, 0)),
                  pl.BlockSpec((1, m), lambda i: (0, 0))],
        out_specs=[pl.BlockSpec((_BR, m), lambda i: (i, 0)),
                   pl.BlockSpec((_BR, m), lambda i: (i, 0))],
        out_shape=[jax.ShapeDtypeStruct((n, m), jnp.float32),
                   jax.ShapeDtypeStruct((n_p, m), jnp.float32)],
    )(X0, m0, m1, dinv_b, Wd, Wbr, b.reshape(1, m))


def _zbranch_body(a_ref, ws1_ref, bs1_ref, ws2_ref, bs2_ref, w3a_ref,
                  w3b_ref, b3_ref, o_ref):
    h02 = jnp.dot(a_ref[...], ws1_ref[...],
                  preferred_element_type=jnp.float32) + bs1_ref[...]
    h12 = jnp.dot(h02, ws2_ref[...],
                  preferred_element_type=jnp.float32) + bs2_ref[...]
    x = jnp.dot(h02, w3a_ref[...], preferred_element_type=jnp.float32)
    x = x + jnp.dot(h12, w3b_ref[...], preferred_element_type=jnp.float32)
    o_ref[...] = x + b3_ref[...]


def _zbranch(hz, Ws1, bs1, Ws2, bs2, W3a, W3b, b3):
    n, k = hz.shape
    m = W3a.shape[1]
    grid = (pl.cdiv(n, _BR),)
    full = lambda r, c: pl.BlockSpec((r, c), lambda i: (0, 0))
    return pl.pallas_call(
        _zbranch_body,
        grid=grid,
        in_specs=[
            pl.BlockSpec((_BR, k), lambda i: (i, 0)),
            full(k, m), full(1, m), full(m, m), full(1, m),
            full(m, m), full(m, m), full(1, m),
        ],
        out_specs=pl.BlockSpec((_BR, m), lambda i: (i, 0)),
        out_shape=jax.ShapeDtypeStruct((n, m), jnp.float32),
    )(hz, Ws1, bs1.reshape(1, -1), Ws2, bs2.reshape(1, -1),
      W3a, W3b, b3.reshape(1, -1))


def _final3_body(a_ref, b_ref, c_ref, w_ref, bias_ref, o_ref):
    x = a_ref[...] + b_ref[...] + c_ref[...]
    x = jnp.where(x >= 0, x, 0.01 * x)
    o_ref[...] = jnp.dot(x, w_ref[...],
                         preferred_element_type=jnp.float32) + bias_ref[...]


def _final3(h1, h2, h3, W, b):
    n, k = h1.shape
    m = W.shape[1]
    grid = (pl.cdiv(n, _BR),)
    return pl.pallas_call(
        _final3_body,
        grid=grid,
        in_specs=[
            pl.BlockSpec((_BR, k), lambda i: (i, 0)),
            pl.BlockSpec((_BR, k), lambda i: (i, 0)),
            pl.BlockSpec((_BR, k), lambda i: (i, 0)),
            pl.BlockSpec((k, m), lambda i: (0, 0)),
            pl.BlockSpec((1, m), lambda i: (0, 0)),
        ],
        out_specs=pl.BlockSpec((_BR, m), lambda i: (i, 0)),
        out_shape=jax.ShapeDtypeStruct((n, m), jnp.float32),
    )(h1, h2, h3, W, b.reshape(1, m))


def _mm2(A, B, W1, W2, b, act=None):
    n, k = A.shape
    m = W1.shape[1]
    grid = (pl.cdiv(n, _BR),)
    return pl.pallas_call(
        functools.partial(_mm2_body, act=act),
        grid=grid,
        in_specs=[
            pl.BlockSpec((_BR, k), lambda i: (i, 0)),
            pl.BlockSpec((_BR, B.shape[1]), lambda i: (i, 0)),
            pl.BlockSpec((W1.shape[0], m), lambda i: (0, 0)),
            pl.BlockSpec((W2.shape[0], m), lambda i: (0, 0)),
            pl.BlockSpec((1, m), lambda i: (0, 0)),
        ],
        out_specs=pl.BlockSpec((_BR, m), lambda i: (i, 0)),
        out_shape=jax.ShapeDtypeStruct((n, m), jnp.float32),
    )(A, B, W1, W2, b.reshape(1, m))


def _chunks(total):
    out = []
    off = 0
    while off < total:
        size = _STG if total - off >= _STG else 8
        out.append((off, size))
        off += size
    return out


def _zero_stage_2d(stg_v, rows, width):
    def zrow(j, _):
        for kk in range(width // 16):
            stg_v[j, pl.ds(kk * 16, 16)] = jnp.zeros((16,), jnp.float32)
        return ()

    lax.fori_loop(0, rows, zrow, ())


@functools.lru_cache(maxsize=None)
def _make_deg_kernel(n_ps, e_pads):
    mesh = plsc.VectorSubcoreMesh(
        core_axis_name="c", subcore_axis_name="s", num_cores=_NC,
        num_subcores=_NS)
    nbs = tuple(e // (_NS * _EB) for e in e_pads)
    scratch = [pltpu.VMEM((max(nbs), _EB), jnp.int32),
               pltpu.VMEM((_EB,), jnp.float32),
               pltpu.VMEM((512,), jnp.float32)]
    for n_p in n_ps:
        scratch.append(pltpu.VMEM_SHARED((n_p,), jnp.float32))
    out_type = tuple(jax.ShapeDtypeStruct((n_p,), jnp.float32) for n_p in n_ps)

    @functools.partial(pl.kernel, out_type=out_type, mesh=mesh,
                       scratch_types=scratch)
    def deg_kernel(sd0, sd1, sd2, o0, o1, o2,
                   idx_v, ones_v, stg_v, a0, a1, a2):
        c = lax.axis_index("c")
        s = lax.axis_index("s")
        for i in range(8):
            ones_v[pl.ds(i * 16, 16)] = jnp.full((16,), 1.0, jnp.float32)

        def do_graph(sd, out, acc, n_p, nb):
            rps = n_p // _NS
            base = s * rps

            def zs(j, _):
                stg_v[pl.ds(j * 16, 16)] = jnp.zeros((16,), jnp.float32)
                return ()

            lax.fori_loop(0, 32, zs, ())
            for off, size in _chunks(rps):
                pltpu.sync_copy(stg_v.at[pl.ds(0, min(size, 512))],
                                acc.at[pl.ds(base + off, min(size, 512))])
            pltpu.sync_copy(sd.at[pl.ds(s * nb, nb)], idx_v.at[pl.ds(0, nb)])
            plsc.subcore_barrier()

            def body(j, _):
                pltpu.sync_copy(ones_v, acc.at[idx_v.at[j]], add=True)
                return ()

            lax.fori_loop(0, nb, body, ())
            plsc.subcore_barrier()
            for off, size in _chunks(rps):
                sz = min(size, 512)
                pltpu.sync_copy(acc.at[pl.ds(base + off, sz)],
                                stg_v.at[pl.ds(0, sz)])
                pltpu.sync_copy(stg_v.at[pl.ds(0, sz)],
                                out.at[pl.ds(base + off, sz)])

        @pl.when(c == 0)
        def _():
            do_graph(sd0, o0, a0, n_ps[0], nbs[0])
            do_graph(sd1, o1, a1, n_ps[1], nbs[1])

        @pl.when(c == 1)
        def _():
            do_graph(sd2, o2, a2, n_ps[2], nbs[2])

    return deg_kernel


@functools.lru_cache(maxsize=None)
def _make_spmm_kernel(n_p, e_pad, width):
    nblk = e_pad // (_NC * _NS * _EB)
    rps = n_p // _NS
    mesh = plsc.VectorSubcoreMesh(
        core_axis_name="c", subcore_axis_name="s", num_cores=_NC,
        num_subcores=_NS)

    @functools.partial(
        pl.kernel,
        out_type=jax.ShapeDtypeStruct((_NC, n_p, width), jnp.float32),
        mesh=mesh,
        scratch_types=[
            pltpu.VMEM((nblk, _EB), jnp.int32),
            pltpu.VMEM((nblk, _EB), jnp.int32),
            pltpu.VMEM((_EB, width), jnp.float32),
            pltpu.VMEM((_EB, width), jnp.float32),
            pltpu.VMEM_SHARED((n_p, width), jnp.float32),
            pltpu.SemaphoreType.DMA,
            pltpu.SemaphoreType.DMA,
        ])
    def spmm_kernel(xs, ss2d, sd2d, out, src_v, dst_v, rows_a, rows_b,
                    acc, sem_a, sem_b):
        c = lax.axis_index("c")
        s = lax.axis_index("s")
        w = c * _NS + s
        base = s * rps
        _zero_stage_2d(rows_a, _STG, width)
        pltpu.sync_copy(ss2d.at[pl.ds(w * nblk, nblk)], src_v)
        pltpu.sync_copy(sd2d.at[pl.ds(w * nblk, nblk)], dst_v)
        for off, size in _chunks(rps):
            pltpu.sync_copy(rows_a.at[pl.ds(0, size)],
                            acc.at[pl.ds(base + off, size)])
        plsc.subcore_barrier()

        def gath(j, buf, sem):
            pltpu.async_copy(xs.at[src_v.at[j]], buf, sem)

        def gwait(j, buf, sem):
            pltpu.make_async_copy(xs.at[src_v.at[j]], buf, sem).wait()

        def scat(j, buf):
            pltpu.sync_copy(buf, acc.at[dst_v.at[j]], add=True)

        nblk2 = nblk // 2

        def body(i, _):
            ja, jb = 2 * i, 2 * i + 1

            @pl.when(i == 0)
            def _():
                gath(ja, rows_a, sem_a)

            gath(jb, rows_b, sem_b)
            gwait(ja, rows_a, sem_a)
            scat(ja, rows_a)

            @pl.when(i < nblk2 - 1)
            def _():
                gath(jb + 1, rows_a, sem_a)

            gwait(jb, rows_b, sem_b)
            scat(jb, rows_b)
            return ()

        lax.fori_loop(0, nblk2, body, ())
        plsc.subcore_barrier()
        for off, size in _chunks(rps):
            pltpu.sync_copy(acc.at[pl.ds(base + off, size)],
                            rows_a.at[pl.ds(0, size)])
            pltpu.sync_copy(rows_a.at[pl.ds(0, size)],
                            out.at[c, pl.ds(base + off, size)])

    return spmm_kernel


def _pad_graph(ss, sd, n):
    e = ss.shape[0]
    n_p = (n // 128 + 2) * 128
    e_pad = -(-e // (8 * _NC * _NS * _EB)) * (8 * _NC * _NS * _EB)
    pad = e_pad - e
    scratch_rows = n_p - n
    pad_idx = n + (jnp.arange(pad, dtype=jnp.int32) % scratch_rows)
    ss_p = jnp.concatenate([ss, pad_idx])
    sd_p = jnp.concatenate([sd, pad_idx])
    return n_p, e_pad, ss_p, sd_p


def _graph_stage(h, inz, iz, lam, deg, n_p, e_pad, ss_p, sd_p,
                 W_c1, b_c1, W_c2, b_c2, W_sg1, b_sg1, W_sg2, b_sg2,
                 W_lin3, b_lin3):
    n = inz.shape[0]
    D = h.shape[1]
    dinv = jnp.clip(deg[:n], 1.0, None) ** -0.5
    dinv_b = jnp.broadcast_to(dinv[:, None], (n, D))
    r = 2.0 / lam
    h_nz = h[inz]
    h_z = h[iz]

    sd2d = sd_p.reshape(-1, _EB)
    ss2d = ss_p.reshape(-1, _EB)
    spmm = _make_spmm_kernel(n_p, e_pad, D)

    def cheb(X0, xs, W, b):
        msg2 = spmm(xs, ss2d, sd2d)
        return _cheb_mm(X0, msg2[0], msg2[1], dinv_b,
                        W[:D] - W[D:], r * W[D:], b, n_p)

    xs0 = jnp.pad(h_nz * dinv[:, None], ((0, n_p - n), (0, 0)))
    h01, xs1 = cheb(h_nz, xs0, W_c1, b_c1)
    h11, _ = cheb(h01, xs1, W_c2, b_c2)
    top = _mm2(h01, h11, W_lin3[:D], W_lin3[D:], b_lin3)
    bot = _zbranch(h_z, W_sg1, b_sg1, W_sg2, b_sg2,
                   W_lin3[:D], W_lin3[D:], b_lin3)
    return jnp.concatenate([top, bot], axis=0)


def kernel(in_feat, W_lin, b_lin, W_lin2, b_lin2, W_sg1, b_sg1, W_sg2, b_sg2,
           W_lin3, b_lin3, W_lin4, b_lin4, W_c1, b_c1, W_c2, b_c2,
           sub_src0, sub_dst0, idx_nz0, idx_z0, lam0,
           sub_src1, sub_dst1, idx_nz1, idx_z1, lam1,
           sub_src2, sub_dst2, idx_nz2, idx_z2, lam2):
    graphs = [(sub_src0, sub_dst0, idx_nz0, idx_z0, lam0),
              (sub_src1, sub_dst1, idx_nz1, idx_z1, lam1),
              (sub_src2, sub_dst2, idx_nz2, idx_z2, lam2)]

    pads = [_pad_graph(ss, sd, inz.shape[0])
            for ss, sd, inz, iz, lam in graphs]
    n_ps = tuple(p[0] for p in pads)
    e_pads = tuple(p[1] for p in pads)

    degk = _make_deg_kernel(n_ps, e_pads)
    degs = degk(pads[0][3].reshape(-1, _EB), pads[1][3].reshape(-1, _EB),
                pads[2][3].reshape(-1, _EB))

    h = _mm(in_feat, W_lin, b_lin, act="leaky")
    h = _mm(h, W_lin2, b_lin2, act="leaky")

    hs = []
    for g, (ss, sd, inz, iz, lam) in enumerate(graphs):
        n_p, e_pad, ss_p, sd_p = pads[g]
        h = _graph_stage(h, inz, iz, lam, degs[g], n_p, e_pad, ss_p, sd_p,
                         W_c1, b_c1, W_c2, b_c2, W_sg1, b_sg1, W_sg2, b_sg2,
                         W_lin3, b_lin3)
        hs.append(h)

    NCo = W_lin4.shape[1]
    W4p = jnp.zeros((W_lin4.shape[0], 128), jnp.float32).at[:, :NCo].set(W_lin4)
    b4p = jnp.zeros((128,), jnp.float32).at[:NCo].set(b_lin4)
    out = _final3(hs[0], hs[1], hs[2], W4p, b4p)
    return out[:, :NCo]


# speedup vs baseline: 18.7021x; 1.0011x over previous
import functools

import jax
import jax.numpy as jnp
from jax import lax
from jax.experimental import pallas as pl
from jax.experimental.pallas import tpu as pltpu
import jax.experimental.pallas.tpu_sc as plsc

_NC = 2
_NS = 16
_EB = 128
_STG = 128


_BR = 1024


def _act(x, act):
    if act == "leaky":
        return jnp.where(x >= 0, x, 0.01 * x)
    if act == "relu":
        return jnp.maximum(x, 0.0)
    return x


def _mm1_body(a_ref, w_ref, b_ref, o_ref, *, act):
    x = jnp.dot(a_ref[...], w_ref[...], preferred_element_type=jnp.float32)
    o_ref[...] = _act(x + b_ref[...], act)


def _mm2_body(a_ref, b2_ref, w1_ref, w2_ref, b_ref, o_ref, *, act):
    x = jnp.dot(a_ref[...], w1_ref[...], preferred_element_type=jnp.float32)
    x = x + jnp.dot(b2_ref[...], w2_ref[...], preferred_element_type=jnp.float32)
    o_ref[...] = _act(x + b_ref[...], act)


def _mm(A, W, b, act=None):
    n, k = A.shape
    ko, m = W.shape
    grid = (pl.cdiv(n, _BR),)
    return pl.pallas_call(
        functools.partial(_mm1_body, act=act),
        grid=grid,
        in_specs=[
            pl.BlockSpec((_BR, k), lambda i: (i, 0)),
            pl.BlockSpec((ko, m), lambda i: (0, 0)),
            pl.BlockSpec((1, m), lambda i: (0, 0)),
        ],
        out_specs=pl.BlockSpec((_BR, m), lambda i: (i, 0)),
        out_shape=jax.ShapeDtypeStruct((n, m), jnp.float32),
    )(A, W, b.reshape(1, m))


def _mm3_body(a_ref, b2_ref, c3_ref, w1_ref, w2_ref, w3_ref, b_ref, o_ref, *, act):
    x = jnp.dot(a_ref[...], w1_ref[...], preferred_element_type=jnp.float32)
    x = x + jnp.dot(b2_ref[...], w2_ref[...], preferred_element_type=jnp.float32)
    x = x + jnp.dot(c3_ref[...], w3_ref[...], preferred_element_type=jnp.float32)
    o_ref[...] = _act(x + b_ref[...], act)


def _mm3(A, B, C, W1, W2, W3, b, act=None):
    n, k = A.shape
    m = W1.shape[1]
    grid = (pl.cdiv(n, _BR),)
    return pl.pallas_call(
        functools.partial(_mm3_body, act=act),
        grid=grid,
        in_specs=[
            pl.BlockSpec((_BR, k), lambda i: (i, 0)),
            pl.BlockSpec((_BR, B.shape[1]), lambda i: (i, 0)),
            pl.BlockSpec((_BR, C.shape[1]), lambda i: (i, 0)),
            pl.BlockSpec((W1.shape[0], m), lambda i: (0, 0)),
            pl.BlockSpec((W2.shape[0], m), lambda i: (0, 0)),
            pl.BlockSpec((W3.shape[0], m), lambda i: (0, 0)),
            pl.BlockSpec((1, m), lambda i: (0, 0)),
        ],
        out_specs=pl.BlockSpec((_BR, m), lambda i: (i, 0)),
        out_shape=jax.ShapeDtypeStruct((n, m), jnp.float32),
    )(A, B, C, W1, W2, W3, b.reshape(1, m))


def _cheb_body(x0_ref, m0_ref, m1_ref, dv_ref, wd_ref, wbr_ref, b_ref,
               o_ref, xs_ref):
    x1s = dv_ref[...] * (m0_ref[...] + m1_ref[...])
    x = jnp.dot(x0_ref[...], wd_ref[...], preferred_element_type=jnp.float32)
    x = x + jnp.dot(x1s, wbr_ref[...], preferred_element_type=jnp.float32)
    o = jnp.maximum(x + b_ref[...], 0.0)
    o_ref[...] = o
    xs_ref[...] = dv_ref[...] * o


def _cheb_mm(X0, m0, m1, dinv_b, Wd, Wbr, b, n_p):
    n, k = X0.shape
    m = Wd.shape[1]
    grid = (pl.cdiv(n, _BR),)
    row = pl.BlockSpec((_BR, k), lambda i: (i, 0))
    return pl.pallas_call(
        _cheb_body,
        grid=grid,
        in_specs=[row, row, row, row,
                  pl.BlockSpec((k, m), lambda i: (0, 0)),
                  pl.BlockSpec((k, m), lambda i: (0, 0)),
                  pl.BlockSpec((1, m), lambda i: (0, 0))],
        out_specs=[pl.BlockSpec((_BR, m), lambda i: (i, 0)),
                   pl.BlockSpec((_BR, m), lambda i: (i, 0))],
        out_shape=[jax.ShapeDtypeStruct((n, m), jnp.float32),
                   jax.ShapeDtypeStruct((n_p, m), jnp.float32)],
    )(X0, m0, m1, dinv_b, Wd, Wbr, b.reshape(1, m))


def _zbranch_body(a_ref, ws1_ref, bs1_ref, ws2_ref, bs2_ref, w3a_ref,
                  w3b_ref, b3_ref, o_ref):
    h02 = jnp.dot(a_ref[...], ws1_ref[...],
                  preferred_element_type=jnp.float32) + bs1_ref[...]
    h12 = jnp.dot(h02, ws2_ref[...],
                  preferred_element_type=jnp.float32) + bs2_ref[...]
    x = jnp.dot(h02, w3a_ref[...], preferred_element_type=jnp.float32)
    x = x + jnp.dot(h12, w3b_ref[...], preferred_element_type=jnp.float32)
    o_ref[...] = x + b3_ref[...]


def _zbranch(hz, Ws1, bs1, Ws2, bs2, W3a, W3b, b3):
    n, k = hz.shape
    m = W3a.shape[1]
    grid = (pl.cdiv(n, _BR),)
    full = lambda r, c: pl.BlockSpec((r, c), lambda i: (0, 0))
    return pl.pallas_call(
        _zbranch_body,
        grid=grid,
        in_specs=[
            pl.BlockSpec((_BR, k), lambda i: (i, 0)),
            full(k, m), full(1, m), full(m, m), full(1, m),
            full(m, m), full(m, m), full(1, m),
        ],
        out_specs=pl.BlockSpec((_BR, m), lambda i: (i, 0)),
        out_shape=jax.ShapeDtypeStruct((n, m), jnp.float32),
    )(hz, Ws1, bs1.reshape(1, -1), Ws2, bs2.reshape(1, -1),
      W3a, W3b, b3.reshape(1, -1))


def _final3_body(a_ref, b_ref, c_ref, w_ref, bias_ref, o_ref):
    x = a_ref[...] + b_ref[...] + c_ref[...]
    x = jnp.where(x >= 0, x, 0.01 * x)
    o_ref[...] = jnp.dot(x, w_ref[...],
                         preferred_element_type=jnp.float32) + bias_ref[...]


def _final3(h1, h2, h3, W, b):
    n, k = h1.shape
    m = W.shape[1]
    grid = (pl.cdiv(n, _BR),)
    return pl.pallas_call(
        _final3_body,
        grid=grid,
        in_specs=[
            pl.BlockSpec((_BR, k), lambda i: (i, 0)),
            pl.BlockSpec((_BR, k), lambda i: (i, 0)),
            pl.BlockSpec((_BR, k), lambda i: (i, 0)),
            pl.BlockSpec((k, m), lambda i: (0, 0)),
            pl.BlockSpec((1, m), lambda i: (0, 0)),
        ],
        out_specs=pl.BlockSpec((_BR, m), lambda i: (i, 0)),
        out_shape=jax.ShapeDtypeStruct((n, m), jnp.float32),
    )(h1, h2, h3, W, b.reshape(1, m))


def _mm2(A, B, W1, W2, b, act=None):
    n, k = A.shape
    m = W1.shape[1]
    grid = (pl.cdiv(n, _BR),)
    return pl.pallas_call(
        functools.partial(_mm2_body, act=act),
        grid=grid,
        in_specs=[
            pl.BlockSpec((_BR, k), lambda i: (i, 0)),
            pl.BlockSpec((_BR, B.shape[1]), lambda i: (i, 0)),
            pl.BlockSpec((W1.shape[0], m), lambda i: (0, 0)),
            pl.BlockSpec((W2.shape[0], m), lambda i: (0, 0)),
            pl.BlockSpec((1, m), lambda i: (0, 0)),
        ],
        out_specs=pl.BlockSpec((_BR, m), lambda i: (i, 0)),
        out_shape=jax.ShapeDtypeStruct((n, m), jnp.float32),
    )(A, B, W1, W2, b.reshape(1, m))


def _chunks(total):
    out = []
    off = 0
    while off < total:
        size = _STG if total - off >= _STG else 8
        out.append((off, size))
        off += size
    return out


def _zero_stage_2d(stg_v, rows, width):
    def zrow(j, _):
        for kk in range(width // 16):
            stg_v[j, pl.ds(kk * 16, 16)] = jnp.zeros((16,), jnp.float32)
        return ()

    lax.fori_loop(0, rows, zrow, ())


@functools.lru_cache(maxsize=None)
def _make_deg_kernel(n_ps, e_pads):
    mesh = plsc.VectorSubcoreMesh(
        core_axis_name="c", subcore_axis_name="s", num_cores=_NC,
        num_subcores=_NS)
    nbs = tuple(e // (_NS * _EB) for e in e_pads)
    scratch = [pltpu.VMEM((max(nbs), _EB), jnp.int32),
               pltpu.VMEM((_EB,), jnp.float32),
               pltpu.VMEM((512,), jnp.float32)]
    for n_p in n_ps:
        scratch.append(pltpu.VMEM_SHARED((n_p,), jnp.float32))
    out_type = tuple(jax.ShapeDtypeStruct((n_p,), jnp.float32) for n_p in n_ps)

    @functools.partial(pl.kernel, out_type=out_type, mesh=mesh,
                       scratch_types=scratch)
    def deg_kernel(sd0, sd1, sd2, o0, o1, o2,
                   idx_v, ones_v, stg_v, a0, a1, a2):
        c = lax.axis_index("c")
        s = lax.axis_index("s")
        for i in range(8):
            ones_v[pl.ds(i * 16, 16)] = jnp.full((16,), 1.0, jnp.float32)

        def do_graph(sd, out, acc, n_p, nb):
            rps = n_p // _NS
            base = s * rps

            def zs(j, _):
                stg_v[pl.ds(j * 16, 16)] = jnp.zeros((16,), jnp.float32)
                return ()

            lax.fori_loop(0, 32, zs, ())
            for off, size in _chunks(rps):
                pltpu.sync_copy(stg_v.at[pl.ds(0, min(size, 512))],
                                acc.at[pl.ds(base + off, min(size, 512))])
            pltpu.sync_copy(sd.at[pl.ds(s * nb, nb)], idx_v.at[pl.ds(0, nb)])
            plsc.subcore_barrier()

            def body(j, _):
                pltpu.sync_copy(ones_v, acc.at[idx_v.at[j]], add=True)
                return ()

            lax.fori_loop(0, nb, body, ())
            plsc.subcore_barrier()
            for off, size in _chunks(rps):
                sz = min(size, 512)
                pltpu.sync_copy(acc.at[pl.ds(base + off, sz)],
                                stg_v.at[pl.ds(0, sz)])
                pltpu.sync_copy(stg_v.at[pl.ds(0, sz)],
                                out.at[pl.ds(base + off, sz)])

        @pl.when(c == 0)
        def _():
            do_graph(sd0, o0, a0, n_ps[0], nbs[0])
            do_graph(sd1, o1, a1, n_ps[1], nbs[1])

        @pl.when(c == 1)
        def _():
            do_graph(sd2, o2, a2, n_ps[2], nbs[2])

    return deg_kernel


@functools.lru_cache(maxsize=None)
def _make_spmm_kernel(n_p, e_pad, width):
    nblk = e_pad // (_NC * _NS * _EB)
    rps = n_p // _NS
    mesh = plsc.VectorSubcoreMesh(
        core_axis_name="c", subcore_axis_name="s", num_cores=_NC,
        num_subcores=_NS)

    @functools.partial(
        pl.kernel,
        out_type=jax.ShapeDtypeStruct((_NC, n_p, width), jnp.float32),
        mesh=mesh,
        scratch_types=[
            pltpu.VMEM((nblk, _EB), jnp.int32),
            pltpu.VMEM((nblk, _EB), jnp.int32),
            pltpu.VMEM((_EB, width), jnp.float32),
            pltpu.VMEM((_EB, width), jnp.float32),
            pltpu.VMEM_SHARED((n_p, width), jnp.float32),
            pltpu.SemaphoreType.DMA,
            pltpu.SemaphoreType.DMA,
        ])
    def spmm_kernel(xs, ss2d, sd2d, out, src_v, dst_v, rows_a, rows_b,
                    acc, sem_a, sem_b):
        c = lax.axis_index("c")
        s = lax.axis_index("s")
        w = c * _NS + s
        base = s * rps
        _zero_stage_2d(rows_a, _STG, width)
        pltpu.sync_copy(ss2d.at[pl.ds(w * nblk, nblk)], src_v)
        pltpu.sync_copy(sd2d.at[pl.ds(w * nblk, nblk)], dst_v)
        for off, size in _chunks(rps):
            pltpu.sync_copy(rows_a.at[pl.ds(0, size)],
                            acc.at[pl.ds(base + off, size)])
        plsc.subcore_barrier()

        def gath(j, buf, sem):
            pltpu.async_copy(xs.at[src_v.at[j]], buf, sem)

        def gwait(j, buf, sem):
            pltpu.make_async_copy(xs.at[src_v.at[j]], buf, sem).wait()

        def scat(j, buf):
            pltpu.sync_copy(buf, acc.at[dst_v.at[j]], add=True)

        nblk2 = nblk // 2

        def body(i, _):
            ja, jb = 2 * i, 2 * i + 1

            @pl.when(i == 0)
            def _():
                gath(ja, rows_a, sem_a)

            gath(jb, rows_b, sem_b)
            gwait(ja, rows_a, sem_a)
            scat(ja, rows_a)

            @pl.when(i < nblk2 - 1)
            def _():
                gath(jb + 1, rows_a, sem_a)

            gwait(jb, rows_b, sem_b)
            scat(jb, rows_b)
            return ()

        lax.fori_loop(0, nblk2, body, (), unroll=2)
        plsc.subcore_barrier()
        for off, size in _chunks(rps):
            pltpu.sync_copy(acc.at[pl.ds(base + off, size)],
                            rows_a.at[pl.ds(0, size)])
            pltpu.sync_copy(rows_a.at[pl.ds(0, size)],
                            out.at[c, pl.ds(base + off, size)])

    return spmm_kernel


def _pad_graph(ss, sd, n):
    e = ss.shape[0]
    n_p = (n // 128 + 2) * 128
    e_pad = -(-e // (8 * _NC * _NS * _EB)) * (8 * _NC * _NS * _EB)
    pad = e_pad - e
    scratch_rows = n_p - n
    pad_idx = n + (jnp.arange(pad, dtype=jnp.int32) % scratch_rows)
    ss_p = jnp.concatenate([ss, pad_idx])
    sd_p = jnp.concatenate([sd, pad_idx])
    return n_p, e_pad, ss_p, sd_p


def _graph_stage(h, inz, iz, lam, deg, n_p, e_pad, ss_p, sd_p,
                 W_c1, b_c1, W_c2, b_c2, W_sg1, b_sg1, W_sg2, b_sg2,
                 W_lin3, b_lin3):
    n = inz.shape[0]
    D = h.shape[1]
    dinv = jnp.clip(deg[:n], 1.0, None) ** -0.5
    dinv_b = jnp.broadcast_to(dinv[:, None], (n, D))
    r = 2.0 / lam
    h_nz = h[inz]
    h_z = h[iz]

    sd2d = sd_p.reshape(-1, _EB)
    ss2d = ss_p.reshape(-1, _EB)
    spmm = _make_spmm_kernel(n_p, e_pad, D)

    def cheb(X0, xs, W, b):
        msg2 = spmm(xs, ss2d, sd2d)
        return _cheb_mm(X0, msg2[0], msg2[1], dinv_b,
                        W[:D] - W[D:], r * W[D:], b, n_p)

    xs0 = jnp.pad(h_nz * dinv[:, None], ((0, n_p - n), (0, 0)))
    h01, xs1 = cheb(h_nz, xs0, W_c1, b_c1)
    h11, _ = cheb(h01, xs1, W_c2, b_c2)
    top = _mm2(h01, h11, W_lin3[:D], W_lin3[D:], b_lin3)
    bot = _zbranch(h_z, W_sg1, b_sg1, W_sg2, b_sg2,
                   W_lin3[:D], W_lin3[D:], b_lin3)
    return jnp.concatenate([top, bot], axis=0)


def kernel(in_feat, W_lin, b_lin, W_lin2, b_lin2, W_sg1, b_sg1, W_sg2, b_sg2,
           W_lin3, b_lin3, W_lin4, b_lin4, W_c1, b_c1, W_c2, b_c2,
           sub_src0, sub_dst0, idx_nz0, idx_z0, lam0,
           sub_src1, sub_dst1, idx_nz1, idx_z1, lam1,
           sub_src2, sub_dst2, idx_nz2, idx_z2, lam2):
    graphs = [(sub_src0, sub_dst0, idx_nz0, idx_z0, lam0),
              (sub_src1, sub_dst1, idx_nz1, idx_z1, lam1),
              (sub_src2, sub_dst2, idx_nz2, idx_z2, lam2)]

    pads = [_pad_graph(ss, sd, inz.shape[0])
            for ss, sd, inz, iz, lam in graphs]
    n_ps = tuple(p[0] for p in pads)
    e_pads = tuple(p[1] for p in pads)

    degk = _make_deg_kernel(n_ps, e_pads)
    degs = degk(pads[0][3].reshape(-1, _EB), pads[1][3].reshape(-1, _EB),
                pads[2][3].reshape(-1, _EB))

    h = _mm(in_feat, W_lin, b_lin, act="leaky")
    h = _mm(h, W_lin2, b_lin2, act="leaky")

    hs = []
    for g, (ss, sd, inz, iz, lam) in enumerate(graphs):
        n_p, e_pad, ss_p, sd_p = pads[g]
        h = _graph_stage(h, inz, iz, lam, degs[g], n_p, e_pad, ss_p, sd_p,
                         W_c1, b_c1, W_c2, b_c2, W_sg1, b_sg1, W_sg2, b_sg2,
                         W_lin3, b_lin3)
        hs.append(h)

    NCo = W_lin4.shape[1]
    W4p = jnp.zeros((W_lin4.shape[0], 128), jnp.float32).at[:, :NCo].set(W_lin4)
    b4p = jnp.zeros((128,), jnp.float32).at[:NCo].set(b_lin4)
    out = _final3(hs[0], hs[1], hs[2], W4p, b4p)
    return out[:, :NCo]


# speedup vs baseline: 18.9553x; 1.0135x over previous
import functools

import jax
import jax.numpy as jnp
from jax import lax
from jax.experimental import pallas as pl
from jax.experimental.pallas import tpu as pltpu
import jax.experimental.pallas.tpu_sc as plsc

_NC = 2
_NS = 16
_EB = 128
_STG = 128


_BR = 1024


def _act(x, act):
    if act == "leaky":
        return jnp.where(x >= 0, x, 0.01 * x)
    if act == "relu":
        return jnp.maximum(x, 0.0)
    return x


def _mm1_body(a_ref, w_ref, b_ref, o_ref, *, act):
    x = jnp.dot(a_ref[...], w_ref[...], preferred_element_type=jnp.float32)
    o_ref[...] = _act(x + b_ref[...], act)


def _mm2_body(a_ref, b2_ref, w1_ref, w2_ref, b_ref, o_ref, *, act):
    x = jnp.dot(a_ref[...], w1_ref[...], preferred_element_type=jnp.float32)
    x = x + jnp.dot(b2_ref[...], w2_ref[...], preferred_element_type=jnp.float32)
    o_ref[...] = _act(x + b_ref[...], act)


def _mm(A, W, b, act=None):
    n, k = A.shape
    ko, m = W.shape
    grid = (pl.cdiv(n, _BR),)
    return pl.pallas_call(
        functools.partial(_mm1_body, act=act),
        grid=grid,
        in_specs=[
            pl.BlockSpec((_BR, k), lambda i: (i, 0)),
            pl.BlockSpec((ko, m), lambda i: (0, 0)),
            pl.BlockSpec((1, m), lambda i: (0, 0)),
        ],
        out_specs=pl.BlockSpec((_BR, m), lambda i: (i, 0)),
        out_shape=jax.ShapeDtypeStruct((n, m), jnp.float32),
    )(A, W, b.reshape(1, m))


def _mm3_body(a_ref, b2_ref, c3_ref, w1_ref, w2_ref, w3_ref, b_ref, o_ref, *, act):
    x = jnp.dot(a_ref[...], w1_ref[...], preferred_element_type=jnp.float32)
    x = x + jnp.dot(b2_ref[...], w2_ref[...], preferred_element_type=jnp.float32)
    x = x + jnp.dot(c3_ref[...], w3_ref[...], preferred_element_type=jnp.float32)
    o_ref[...] = _act(x + b_ref[...], act)


def _mm3(A, B, C, W1, W2, W3, b, act=None):
    n, k = A.shape
    m = W1.shape[1]
    grid = (pl.cdiv(n, _BR),)
    return pl.pallas_call(
        functools.partial(_mm3_body, act=act),
        grid=grid,
        in_specs=[
            pl.BlockSpec((_BR, k), lambda i: (i, 0)),
            pl.BlockSpec((_BR, B.shape[1]), lambda i: (i, 0)),
            pl.BlockSpec((_BR, C.shape[1]), lambda i: (i, 0)),
            pl.BlockSpec((W1.shape[0], m), lambda i: (0, 0)),
            pl.BlockSpec((W2.shape[0], m), lambda i: (0, 0)),
            pl.BlockSpec((W3.shape[0], m), lambda i: (0, 0)),
            pl.BlockSpec((1, m), lambda i: (0, 0)),
        ],
        out_specs=pl.BlockSpec((_BR, m), lambda i: (i, 0)),
        out_shape=jax.ShapeDtypeStruct((n, m), jnp.float32),
    )(A, B, C, W1, W2, W3, b.reshape(1, m))


def _cheb_body(x0_ref, m0_ref, m1_ref, dv_ref, wd_ref, wbr_ref, b_ref,
               o_ref, xs_ref):
    x1s = dv_ref[...] * (m0_ref[...] + m1_ref[...])
    x = jnp.dot(x0_ref[...], wd_ref[...], preferred_element_type=jnp.float32)
    x = x + jnp.dot(x1s, wbr_ref[...], preferred_element_type=jnp.float32)
    o = jnp.maximum(x + b_ref[...], 0.0)
    o_ref[...] = o
    xs_ref[...] = dv_ref[...] * o


def _cheb_mm(X0, m0, m1, dinv_b, Wd, Wbr, b, n_p):
    n, k = X0.shape
    m = Wd.shape[1]
    grid = (pl.cdiv(n, _BR),)
    row = pl.BlockSpec((_BR, k), lambda i: (i, 0))
    return pl.pallas_call(
        _cheb_body,
        grid=grid,
        in_specs=[row, row, row, row,
                  pl.BlockSpec((k, m), lambda i: (0, 0)),
                  pl.BlockSpec((k, m), lambda i: (0, 0)),
                  pl.BlockSpec((1, m), lambda i: (0, 0))],
        out_specs=[pl.BlockSpec((_BR, m), lambda i: (i, 0)),
                   pl.BlockSpec((_BR, m), lambda i: (i, 0))],
        out_shape=[jax.ShapeDtypeStruct((n, m), jnp.float32),
                   jax.ShapeDtypeStruct((n_p, m), jnp.float32)],
    )(X0, m0, m1, dinv_b, Wd, Wbr, b.reshape(1, m))


def _zbranch_body(a_ref, ws1_ref, bs1_ref, ws2_ref, bs2_ref, w3a_ref,
                  w3b_ref, b3_ref, o_ref):
    h02 = jnp.dot(a_ref[...], ws1_ref[...],
                  preferred_element_type=jnp.float32) + bs1_ref[...]
    h12 = jnp.dot(h02, ws2_ref[...],
                  preferred_element_type=jnp.float32) + bs2_ref[...]
    x = jnp.dot(h02, w3a_ref[...], preferred_element_type=jnp.float32)
    x = x + jnp.dot(h12, w3b_ref[...], preferred_element_type=jnp.float32)
    o_ref[...] = x + b3_ref[...]


def _zbranch(hz, Ws1, bs1, Ws2, bs2, W3a, W3b, b3):
    n, k = hz.shape
    m = W3a.shape[1]
    grid = (pl.cdiv(n, _BR),)
    full = lambda r, c: pl.BlockSpec((r, c), lambda i: (0, 0))
    return pl.pallas_call(
        _zbranch_body,
        grid=grid,
        in_specs=[
            pl.BlockSpec((_BR, k), lambda i: (i, 0)),
            full(k, m), full(1, m), full(m, m), full(1, m),
            full(m, m), full(m, m), full(1, m),
        ],
        out_specs=pl.BlockSpec((_BR, m), lambda i: (i, 0)),
        out_shape=jax.ShapeDtypeStruct((n, m), jnp.float32),
    )(hz, Ws1, bs1.reshape(1, -1), Ws2, bs2.reshape(1, -1),
      W3a, W3b, b3.reshape(1, -1))


def _final3_body(a_ref, b_ref, c_ref, w_ref, bias_ref, o_ref):
    x = a_ref[...] + b_ref[...] + c_ref[...]
    x = jnp.where(x >= 0, x, 0.01 * x)
    o_ref[...] = jnp.dot(x, w_ref[...],
                         preferred_element_type=jnp.float32) + bias_ref[...]


def _final3(h1, h2, h3, W, b):
    n, k = h1.shape
    m = W.shape[1]
    grid = (pl.cdiv(n, _BR),)
    return pl.pallas_call(
        _final3_body,
        grid=grid,
        in_specs=[
            pl.BlockSpec((_BR, k), lambda i: (i, 0)),
            pl.BlockSpec((_BR, k), lambda i: (i, 0)),
            pl.BlockSpec((_BR, k), lambda i: (i, 0)),
            pl.BlockSpec((k, m), lambda i: (0, 0)),
            pl.BlockSpec((1, m), lambda i: (0, 0)),
        ],
        out_specs=pl.BlockSpec((_BR, m), lambda i: (i, 0)),
        out_shape=jax.ShapeDtypeStruct((n, m), jnp.float32),
    )(h1, h2, h3, W, b.reshape(1, m))


def _mm2(A, B, W1, W2, b, act=None):
    n, k = A.shape
    m = W1.shape[1]
    grid = (pl.cdiv(n, _BR),)
    return pl.pallas_call(
        functools.partial(_mm2_body, act=act),
        grid=grid,
        in_specs=[
            pl.BlockSpec((_BR, k), lambda i: (i, 0)),
            pl.BlockSpec((_BR, B.shape[1]), lambda i: (i, 0)),
            pl.BlockSpec((W1.shape[0], m), lambda i: (0, 0)),
            pl.BlockSpec((W2.shape[0], m), lambda i: (0, 0)),
            pl.BlockSpec((1, m), lambda i: (0, 0)),
        ],
        out_specs=pl.BlockSpec((_BR, m), lambda i: (i, 0)),
        out_shape=jax.ShapeDtypeStruct((n, m), jnp.float32),
    )(A, B, W1, W2, b.reshape(1, m))


def _chunks(total):
    out = []
    off = 0
    while off < total:
        size = _STG if total - off >= _STG else 8
        out.append((off, size))
        off += size
    return out


def _zero_stage_2d(stg_v, rows, width):
    def zrow(j, _):
        for kk in range(width // 16):
            stg_v[j, pl.ds(kk * 16, 16)] = jnp.zeros((16,), jnp.float32)
        return ()

    lax.fori_loop(0, rows, zrow, ())


@functools.lru_cache(maxsize=None)
def _make_deg_kernel(n_ps, e_pads):
    mesh = plsc.VectorSubcoreMesh(
        core_axis_name="c", subcore_axis_name="s", num_cores=_NC,
        num_subcores=_NS)
    nbs = tuple(e // (_NS * _EB) for e in e_pads)
    scratch = [pltpu.VMEM((max(nbs), _EB), jnp.int32),
               pltpu.VMEM((_EB,), jnp.float32),
               pltpu.VMEM((512,), jnp.float32)]
    for n_p in n_ps:
        scratch.append(pltpu.VMEM_SHARED((n_p,), jnp.float32))
    out_type = tuple(jax.ShapeDtypeStruct((n_p,), jnp.float32) for n_p in n_ps)

    @functools.partial(pl.kernel, out_type=out_type, mesh=mesh,
                       scratch_types=scratch)
    def deg_kernel(sd0, sd1, sd2, o0, o1, o2,
                   idx_v, ones_v, stg_v, a0, a1, a2):
        c = lax.axis_index("c")
        s = lax.axis_index("s")
        for i in range(8):
            ones_v[pl.ds(i * 16, 16)] = jnp.full((16,), 1.0, jnp.float32)

        def do_graph(sd, out, acc, n_p, nb):
            rps = n_p // _NS
            base = s * rps

            def zs(j, _):
                stg_v[pl.ds(j * 16, 16)] = jnp.zeros((16,), jnp.float32)
                return ()

            lax.fori_loop(0, 32, zs, ())
            for off, size in _chunks(rps):
                pltpu.sync_copy(stg_v.at[pl.ds(0, min(size, 512))],
                                acc.at[pl.ds(base + off, min(size, 512))])
            pltpu.sync_copy(sd.at[pl.ds(s * nb, nb)], idx_v.at[pl.ds(0, nb)])
            plsc.subcore_barrier()

            def body(j, _):
                pltpu.sync_copy(ones_v, acc.at[idx_v.at[j]], add=True)
                return ()

            lax.fori_loop(0, nb, body, ())
            plsc.subcore_barrier()
            for off, size in _chunks(rps):
                sz = min(size, 512)
                pltpu.sync_copy(acc.at[pl.ds(base + off, sz)],
                                stg_v.at[pl.ds(0, sz)])
                pltpu.sync_copy(stg_v.at[pl.ds(0, sz)],
                                out.at[pl.ds(base + off, sz)])

        @pl.when(c == 0)
        def _():
            do_graph(sd0, o0, a0, n_ps[0], nbs[0])
            do_graph(sd1, o1, a1, n_ps[1], nbs[1])

        @pl.when(c == 1)
        def _():
            do_graph(sd2, o2, a2, n_ps[2], nbs[2])

    return deg_kernel


@functools.lru_cache(maxsize=None)
def _make_spmm_kernel(n_p, e_pad, width):
    nblk = e_pad // (_NC * _NS * _EB)
    rps = n_p // _NS
    mesh = plsc.VectorSubcoreMesh(
        core_axis_name="c", subcore_axis_name="s", num_cores=_NC,
        num_subcores=_NS)

    @functools.partial(
        pl.kernel,
        out_type=jax.ShapeDtypeStruct((_NC, n_p, width), jnp.float32),
        mesh=mesh,
        scratch_types=[
            pltpu.VMEM((nblk, _EB), jnp.int32),
            pltpu.VMEM((nblk, _EB), jnp.int32),
            pltpu.VMEM((_EB, width), jnp.float32),
            pltpu.VMEM((_EB, width), jnp.float32),
            pltpu.VMEM_SHARED((n_p, width), jnp.float32),
            pltpu.SemaphoreType.DMA,
            pltpu.SemaphoreType.DMA,
        ])
    def spmm_kernel(xs, ss2d, sd2d, out, src_v, dst_v, rows_a, rows_b,
                    acc, sem_a, sem_b):
        c = lax.axis_index("c")
        s = lax.axis_index("s")
        w = c * _NS + s
        base = s * rps
        _zero_stage_2d(rows_a, _STG, width)
        pltpu.sync_copy(ss2d.at[pl.ds(w * nblk, nblk)], src_v)
        pltpu.sync_copy(sd2d.at[pl.ds(w * nblk, nblk)], dst_v)
        zchunks = _chunks(rps)
        for off, size in zchunks:
            pltpu.async_copy(rows_a.at[pl.ds(0, size)],
                             acc.at[pl.ds(base + off, size)], sem_a)
        for off, size in zchunks:
            pltpu.make_async_copy(rows_a.at[pl.ds(0, size)],
                                  acc.at[pl.ds(base + off, size)],
                                  sem_a).wait()
        plsc.subcore_barrier()

        def gath(j, buf, sem):
            pltpu.async_copy(xs.at[src_v.at[j]], buf, sem)

        def gwait(j, buf, sem):
            pltpu.make_async_copy(xs.at[src_v.at[j]], buf, sem).wait()

        def scat(j, buf):
            pltpu.sync_copy(buf, acc.at[dst_v.at[j]], add=True)

        nblk2 = nblk // 2

        def body(i, _):
            ja, jb = 2 * i, 2 * i + 1

            @pl.when(i == 0)
            def _():
                gath(ja, rows_a, sem_a)

            gath(jb, rows_b, sem_b)
            gwait(ja, rows_a, sem_a)
            scat(ja, rows_a)

            @pl.when(i < nblk2 - 1)
            def _():
                gath(jb + 1, rows_a, sem_a)

            gwait(jb, rows_b, sem_b)
            scat(jb, rows_b)
            return ()

        lax.fori_loop(0, nblk2, body, (), unroll=2)
        plsc.subcore_barrier()
        wchunks = _chunks(rps)
        bufs = (rows_a, rows_b)
        sems = (sem_a, sem_b)
        pending = [None, None]
        for k, (off, size) in enumerate(wchunks):
            t = k % 2
            if pending[t] is not None:
                poff, psize = pending[t]
                pltpu.make_async_copy(
                    bufs[t].at[pl.ds(0, psize)],
                    out.at[c, pl.ds(base + poff, psize)], sems[t]).wait()
            pltpu.sync_copy(acc.at[pl.ds(base + off, size)],
                            bufs[t].at[pl.ds(0, size)])
            pltpu.async_copy(bufs[t].at[pl.ds(0, size)],
                             out.at[c, pl.ds(base + off, size)], sems[t])
            pending[t] = (off, size)
        for t in range(2):
            if pending[t] is not None:
                poff, psize = pending[t]
                pltpu.make_async_copy(
                    bufs[t].at[pl.ds(0, psize)],
                    out.at[c, pl.ds(base + poff, psize)], sems[t]).wait()

    return spmm_kernel


def _pad_graph(ss, sd, n):
    e = ss.shape[0]
    n_p = (n // 128 + 2) * 128
    e_pad = -(-e // (8 * _NC * _NS * _EB)) * (8 * _NC * _NS * _EB)
    pad = e_pad - e
    scratch_rows = n_p - n
    pad_idx = n + (jnp.arange(pad, dtype=jnp.int32) % scratch_rows)
    ss_p = jnp.concatenate([ss, pad_idx])
    sd_p = jnp.concatenate([sd, pad_idx])
    return n_p, e_pad, ss_p, sd_p


def _graph_stage(h, inz, iz, lam, deg, n_p, e_pad, ss_p, sd_p,
                 W_c1, b_c1, W_c2, b_c2, W_sg1, b_sg1, W_sg2, b_sg2,
                 W_lin3, b_lin3):
    n = inz.shape[0]
    D = h.shape[1]
    dinv = jnp.clip(deg[:n], 1.0, None) ** -0.5
    dinv_b = jnp.broadcast_to(dinv[:, None], (n, D))
    r = 2.0 / lam
    h_nz = h[inz]
    h_z = h[iz]

    sd2d = sd_p.reshape(-1, _EB)
    ss2d = ss_p.reshape(-1, _EB)
    spmm = _make_spmm_kernel(n_p, e_pad, D)

    def cheb(X0, xs, W, b):
        msg2 = spmm(xs, ss2d, sd2d)
        return _cheb_mm(X0, msg2[0], msg2[1], dinv_b,
                        W[:D] - W[D:], r * W[D:], b, n_p)

    xs0 = jnp.pad(h_nz * dinv[:, None], ((0, n_p - n), (0, 0)))
    h01, xs1 = cheb(h_nz, xs0, W_c1, b_c1)
    h11, _ = cheb(h01, xs1, W_c2, b_c2)
    top = _mm2(h01, h11, W_lin3[:D], W_lin3[D:], b_lin3)
    bot = _zbranch(h_z, W_sg1, b_sg1, W_sg2, b_sg2,
                   W_lin3[:D], W_lin3[D:], b_lin3)
    return jnp.concatenate([top, bot], axis=0)


def kernel(in_feat, W_lin, b_lin, W_lin2, b_lin2, W_sg1, b_sg1, W_sg2, b_sg2,
           W_lin3, b_lin3, W_lin4, b_lin4, W_c1, b_c1, W_c2, b_c2,
           sub_src0, sub_dst0, idx_nz0, idx_z0, lam0,
           sub_src1, sub_dst1, idx_nz1, idx_z1, lam1,
           sub_src2, sub_dst2, idx_nz2, idx_z2, lam2):
    graphs = [(sub_src0, sub_dst0, idx_nz0, idx_z0, lam0),
              (sub_src1, sub_dst1, idx_nz1, idx_z1, lam1),
              (sub_src2, sub_dst2, idx_nz2, idx_z2, lam2)]

    pads = [_pad_graph(ss, sd, inz.shape[0])
            for ss, sd, inz, iz, lam in graphs]
    n_ps = tuple(p[0] for p in pads)
    e_pads = tuple(p[1] for p in pads)

    degk = _make_deg_kernel(n_ps, e_pads)
    degs = degk(pads[0][3].reshape(-1, _EB), pads[1][3].reshape(-1, _EB),
                pads[2][3].reshape(-1, _EB))

    h = _mm(in_feat, W_lin, b_lin, act="leaky")
    h = _mm(h, W_lin2, b_lin2, act="leaky")

    hs = []
    for g, (ss, sd, inz, iz, lam) in enumerate(graphs):
        n_p, e_pad, ss_p, sd_p = pads[g]
        h = _graph_stage(h, inz, iz, lam, degs[g], n_p, e_pad, ss_p, sd_p,
                         W_c1, b_c1, W_c2, b_c2, W_sg1, b_sg1, W_sg2, b_sg2,
                         W_lin3, b_lin3)
        hs.append(h)

    NCo = W_lin4.shape[1]
    W4p = jnp.zeros((W_lin4.shape[0], 128), jnp.float32).at[:, :NCo].set(W_lin4)
    b4p = jnp.zeros((128,), jnp.float32).at[:NCo].set(b_lin4)
    out = _final3(hs[0], hs[1], hs[2], W4p, b4p)
    return out[:, :NCo]


# speedup vs baseline: 19.1429x; 1.0099x over previous
import functools

import jax
import jax.numpy as jnp
from jax import lax
from jax.experimental import pallas as pl
from jax.experimental.pallas import tpu as pltpu
import jax.experimental.pallas.tpu_sc as plsc

_NC = 2
_NS = 16
_EB = 128
_STG = 128


_BR = 1024


def _act(x, act):
    if act == "leaky":
        return jnp.where(x >= 0, x, 0.01 * x)
    if act == "relu":
        return jnp.maximum(x, 0.0)
    return x


def _mm1_body(a_ref, w_ref, b_ref, o_ref, *, act):
    x = jnp.dot(a_ref[...], w_ref[...], preferred_element_type=jnp.float32)
    o_ref[...] = _act(x + b_ref[...], act)


def _mm2_body(a_ref, b2_ref, w1_ref, w2_ref, b_ref, o_ref, *, act):
    x = jnp.dot(a_ref[...], w1_ref[...], preferred_element_type=jnp.float32)
    x = x + jnp.dot(b2_ref[...], w2_ref[...], preferred_element_type=jnp.float32)
    o_ref[...] = _act(x + b_ref[...], act)


def _mm(A, W, b, act=None):
    n, k = A.shape
    ko, m = W.shape
    grid = (pl.cdiv(n, _BR),)
    return pl.pallas_call(
        functools.partial(_mm1_body, act=act),
        grid=grid,
        in_specs=[
            pl.BlockSpec((_BR, k), lambda i: (i, 0)),
            pl.BlockSpec((ko, m), lambda i: (0, 0)),
            pl.BlockSpec((1, m), lambda i: (0, 0)),
        ],
        out_specs=pl.BlockSpec((_BR, m), lambda i: (i, 0)),
        out_shape=jax.ShapeDtypeStruct((n, m), jnp.float32),
    )(A, W, b.reshape(1, m))


def _mm3_body(a_ref, b2_ref, c3_ref, w1_ref, w2_ref, w3_ref, b_ref, o_ref, *, act):
    x = jnp.dot(a_ref[...], w1_ref[...], preferred_element_type=jnp.float32)
    x = x + jnp.dot(b2_ref[...], w2_ref[...], preferred_element_type=jnp.float32)
    x = x + jnp.dot(c3_ref[...], w3_ref[...], preferred_element_type=jnp.float32)
    o_ref[...] = _act(x + b_ref[...], act)


def _mm3(A, B, C, W1, W2, W3, b, act=None):
    n, k = A.shape
    m = W1.shape[1]
    grid = (pl.cdiv(n, _BR),)
    return pl.pallas_call(
        functools.partial(_mm3_body, act=act),
        grid=grid,
        in_specs=[
            pl.BlockSpec((_BR, k), lambda i: (i, 0)),
            pl.BlockSpec((_BR, B.shape[1]), lambda i: (i, 0)),
            pl.BlockSpec((_BR, C.shape[1]), lambda i: (i, 0)),
            pl.BlockSpec((W1.shape[0], m), lambda i: (0, 0)),
            pl.BlockSpec((W2.shape[0], m), lambda i: (0, 0)),
            pl.BlockSpec((W3.shape[0], m), lambda i: (0, 0)),
            pl.BlockSpec((1, m), lambda i: (0, 0)),
        ],
        out_specs=pl.BlockSpec((_BR, m), lambda i: (i, 0)),
        out_shape=jax.ShapeDtypeStruct((n, m), jnp.float32),
    )(A, B, C, W1, W2, W3, b.reshape(1, m))


def _cheb_body(x0_ref, m0_ref, m1_ref, dv_ref, wd_ref, wbr_ref, b_ref,
               o_ref, xs_ref):
    x1s = dv_ref[...] * (m0_ref[...] + m1_ref[...])
    x = jnp.dot(x0_ref[...], wd_ref[...], preferred_element_type=jnp.float32)
    x = x + jnp.dot(x1s, wbr_ref[...], preferred_element_type=jnp.float32)
    o = jnp.maximum(x + b_ref[...], 0.0)
    o_ref[...] = o
    xs_ref[...] = dv_ref[...] * o


def _cheb_mm(X0, m0, m1, dinv_b, Wd, Wbr, b, n_p):
    n, k = X0.shape
    m = Wd.shape[1]
    grid = (pl.cdiv(n, _BR),)
    row = pl.BlockSpec((_BR, k), lambda i: (i, 0))
    return pl.pallas_call(
        _cheb_body,
        grid=grid,
        in_specs=[row, row, row, row,
                  pl.BlockSpec((k, m), lambda i: (0, 0)),
                  pl.BlockSpec((k, m), lambda i: (0, 0)),
                  pl.BlockSpec((1, m), lambda i: (0, 0))],
        out_specs=[pl.BlockSpec((_BR, m), lambda i: (i, 0)),
                   pl.BlockSpec((_BR, m), lambda i: (i, 0))],
        out_shape=[jax.ShapeDtypeStruct((n, m), jnp.float32),
                   jax.ShapeDtypeStruct((n_p, m), jnp.float32)],
    )(X0, m0, m1, dinv_b, Wd, Wbr, b.reshape(1, m))


def _zbranch_body(a_ref, ws1_ref, bs1_ref, ws2_ref, bs2_ref, w3a_ref,
                  w3b_ref, b3_ref, o_ref):
    h02 = jnp.dot(a_ref[...], ws1_ref[...],
                  preferred_element_type=jnp.float32) + bs1_ref[...]
    h12 = jnp.dot(h02, ws2_ref[...],
                  preferred_element_type=jnp.float32) + bs2_ref[...]
    x = jnp.dot(h02, w3a_ref[...], preferred_element_type=jnp.float32)
    x = x + jnp.dot(h12, w3b_ref[...], preferred_element_type=jnp.float32)
    o_ref[...] = x + b3_ref[...]


def _zbranch(hz, Ws1, bs1, Ws2, bs2, W3a, W3b, b3):
    n, k = hz.shape
    m = W3a.shape[1]
    grid = (pl.cdiv(n, _BR),)
    full = lambda r, c: pl.BlockSpec((r, c), lambda i: (0, 0))
    return pl.pallas_call(
        _zbranch_body,
        grid=grid,
        in_specs=[
            pl.BlockSpec((_BR, k), lambda i: (i, 0)),
            full(k, m), full(1, m), full(m, m), full(1, m),
            full(m, m), full(m, m), full(1, m),
        ],
        out_specs=pl.BlockSpec((_BR, m), lambda i: (i, 0)),
        out_shape=jax.ShapeDtypeStruct((n, m), jnp.float32),
    )(hz, Ws1, bs1.reshape(1, -1), Ws2, bs2.reshape(1, -1),
      W3a, W3b, b3.reshape(1, -1))


def _final3_body(a_ref, b_ref, c_ref, w_ref, bias_ref, o_ref):
    x = a_ref[...] + b_ref[...] + c_ref[...]
    x = jnp.where(x >= 0, x, 0.01 * x)
    o_ref[...] = jnp.dot(x, w_ref[...],
                         preferred_element_type=jnp.float32) + bias_ref[...]


def _final3(h1, h2, h3, W, b):
    n, k = h1.shape
    m = W.shape[1]
    grid = (pl.cdiv(n, _BR),)
    return pl.pallas_call(
        _final3_body,
        grid=grid,
        in_specs=[
            pl.BlockSpec((_BR, k), lambda i: (i, 0)),
            pl.BlockSpec((_BR, k), lambda i: (i, 0)),
            pl.BlockSpec((_BR, k), lambda i: (i, 0)),
            pl.BlockSpec((k, m), lambda i: (0, 0)),
            pl.BlockSpec((1, m), lambda i: (0, 0)),
        ],
        out_specs=pl.BlockSpec((_BR, m), lambda i: (i, 0)),
        out_shape=jax.ShapeDtypeStruct((n, m), jnp.float32),
    )(h1, h2, h3, W, b.reshape(1, m))


def _mm2(A, B, W1, W2, b, act=None):
    n, k = A.shape
    m = W1.shape[1]
    grid = (pl.cdiv(n, _BR),)
    return pl.pallas_call(
        functools.partial(_mm2_body, act=act),
        grid=grid,
        in_specs=[
            pl.BlockSpec((_BR, k), lambda i: (i, 0)),
            pl.BlockSpec((_BR, B.shape[1]), lambda i: (i, 0)),
            pl.BlockSpec((W1.shape[0], m), lambda i: (0, 0)),
            pl.BlockSpec((W2.shape[0], m), lambda i: (0, 0)),
            pl.BlockSpec((1, m), lambda i: (0, 0)),
        ],
        out_specs=pl.BlockSpec((_BR, m), lambda i: (i, 0)),
        out_shape=jax.ShapeDtypeStruct((n, m), jnp.float32),
    )(A, B, W1, W2, b.reshape(1, m))


def _chunks(total):
    out = []
    off = 0
    while total - off >= _STG:
        out.append((off, _STG))
        off += _STG
    if off < total:
        out.append((off, total - off))
    return out


def _zero_stage_2d(stg_v, rows, width):
    def zrow(j, _):
        for kk in range(width // 16):
            stg_v[j, pl.ds(kk * 16, 16)] = jnp.zeros((16,), jnp.float32)
        return ()

    lax.fori_loop(0, rows, zrow, ())


@functools.lru_cache(maxsize=None)
def _make_deg_kernel(n_ps, e_pads):
    mesh = plsc.VectorSubcoreMesh(
        core_axis_name="c", subcore_axis_name="s", num_cores=_NC,
        num_subcores=_NS)
    nbs = tuple(e // (_NS * _EB) for e in e_pads)
    scratch = [pltpu.VMEM((max(nbs), _EB), jnp.int32),
               pltpu.VMEM((_EB,), jnp.float32),
               pltpu.VMEM((512,), jnp.float32)]
    for n_p in n_ps:
        scratch.append(pltpu.VMEM_SHARED((n_p,), jnp.float32))
    out_type = tuple(jax.ShapeDtypeStruct((n_p,), jnp.float32) for n_p in n_ps)

    @functools.partial(pl.kernel, out_type=out_type, mesh=mesh,
                       scratch_types=scratch)
    def deg_kernel(sd0, sd1, sd2, o0, o1, o2,
                   idx_v, ones_v, stg_v, a0, a1, a2):
        c = lax.axis_index("c")
        s = lax.axis_index("s")
        for i in range(8):
            ones_v[pl.ds(i * 16, 16)] = jnp.full((16,), 1.0, jnp.float32)

        def do_graph(sd, out, acc, n_p, nb):
            rps = n_p // _NS
            base = s * rps

            def zs(j, _):
                stg_v[pl.ds(j * 16, 16)] = jnp.zeros((16,), jnp.float32)
                return ()

            lax.fori_loop(0, 32, zs, ())
            for off, size in _chunks(rps):
                pltpu.sync_copy(stg_v.at[pl.ds(0, min(size, 512))],
                                acc.at[pl.ds(base + off, min(size, 512))])
            pltpu.sync_copy(sd.at[pl.ds(s * nb, nb)], idx_v.at[pl.ds(0, nb)])
            plsc.subcore_barrier()

            def body(j, _):
                pltpu.sync_copy(ones_v, acc.at[idx_v.at[j]], add=True)
                return ()

            lax.fori_loop(0, nb, body, ())
            plsc.subcore_barrier()
            for off, size in _chunks(rps):
                sz = min(size, 512)
                pltpu.sync_copy(acc.at[pl.ds(base + off, sz)],
                                stg_v.at[pl.ds(0, sz)])
                pltpu.sync_copy(stg_v.at[pl.ds(0, sz)],
                                out.at[pl.ds(base + off, sz)])

        @pl.when(c == 0)
        def _():
            do_graph(sd0, o0, a0, n_ps[0], nbs[0])
            do_graph(sd1, o1, a1, n_ps[1], nbs[1])

        @pl.when(c == 1)
        def _():
            do_graph(sd2, o2, a2, n_ps[2], nbs[2])

    return deg_kernel


@functools.lru_cache(maxsize=None)
def _make_spmm_kernel(n_p, e_pad, width):
    nblk = e_pad // (_NC * _NS * _EB)
    rps = n_p // _NS
    mesh = plsc.VectorSubcoreMesh(
        core_axis_name="c", subcore_axis_name="s", num_cores=_NC,
        num_subcores=_NS)

    @functools.partial(
        pl.kernel,
        out_type=jax.ShapeDtypeStruct((_NC, n_p, width), jnp.float32),
        mesh=mesh,
        scratch_types=[
            pltpu.VMEM((nblk, _EB), jnp.int32),
            pltpu.VMEM((nblk, _EB), jnp.int32),
            pltpu.VMEM((_EB, width), jnp.float32),
            pltpu.VMEM((_EB, width), jnp.float32),
            pltpu.VMEM_SHARED((n_p, width), jnp.float32),
            pltpu.SemaphoreType.DMA,
            pltpu.SemaphoreType.DMA,
        ])
    def spmm_kernel(xs, ss2d, sd2d, out, src_v, dst_v, rows_a, rows_b,
                    acc, sem_a, sem_b):
        c = lax.axis_index("c")
        s = lax.axis_index("s")
        w = c * _NS + s
        base = s * rps
        _zero_stage_2d(rows_a, _STG, width)
        pltpu.sync_copy(ss2d.at[pl.ds(w * nblk, nblk)], src_v)
        pltpu.sync_copy(sd2d.at[pl.ds(w * nblk, nblk)], dst_v)
        zchunks = _chunks(rps)
        for off, size in zchunks:
            pltpu.async_copy(rows_a.at[pl.ds(0, size)],
                             acc.at[pl.ds(base + off, size)], sem_a)
        for off, size in zchunks:
            pltpu.make_async_copy(rows_a.at[pl.ds(0, size)],
                                  acc.at[pl.ds(base + off, size)],
                                  sem_a).wait()
        plsc.subcore_barrier()

        def gath(j, buf, sem):
            pltpu.async_copy(xs.at[src_v.at[j]], buf, sem)

        def gwait(j, buf, sem):
            pltpu.make_async_copy(xs.at[src_v.at[j]], buf, sem).wait()

        def scat(j, buf):
            pltpu.sync_copy(buf, acc.at[dst_v.at[j]], add=True)

        nblk2 = nblk // 2

        def body(i, _):
            ja, jb = 2 * i, 2 * i + 1

            @pl.when(i == 0)
            def _():
                gath(ja, rows_a, sem_a)

            gath(jb, rows_b, sem_b)
            gwait(ja, rows_a, sem_a)
            scat(ja, rows_a)

            @pl.when(i < nblk2 - 1)
            def _():
                gath(jb + 1, rows_a, sem_a)

            gwait(jb, rows_b, sem_b)
            scat(jb, rows_b)
            return ()

        lax.fori_loop(0, nblk2, body, (), unroll=2)
        plsc.subcore_barrier()
        wchunks = _chunks(rps)
        bufs = (rows_a, rows_b)
        sems = (sem_a, sem_b)
        pending = [None, None]
        for k, (off, size) in enumerate(wchunks):
            t = k % 2
            if pending[t] is not None:
                poff, psize = pending[t]
                pltpu.make_async_copy(
                    bufs[t].at[pl.ds(0, psize)],
                    out.at[c, pl.ds(base + poff, psize)], sems[t]).wait()
            pltpu.sync_copy(acc.at[pl.ds(base + off, size)],
                            bufs[t].at[pl.ds(0, size)])
            pltpu.async_copy(bufs[t].at[pl.ds(0, size)],
                             out.at[c, pl.ds(base + off, size)], sems[t])
            pending[t] = (off, size)
        for t in range(2):
            if pending[t] is not None:
                poff, psize = pending[t]
                pltpu.make_async_copy(
                    bufs[t].at[pl.ds(0, psize)],
                    out.at[c, pl.ds(base + poff, psize)], sems[t]).wait()

    return spmm_kernel


def _pad_graph(ss, sd, n):
    e = ss.shape[0]
    n_p = (n // 128 + 2) * 128
    e_pad = -(-e // (8 * _NC * _NS * _EB)) * (8 * _NC * _NS * _EB)
    pad = e_pad - e
    scratch_rows = n_p - n
    pad_idx = n + (jnp.arange(pad, dtype=jnp.int32) % scratch_rows)
    ss_p = jnp.concatenate([ss, pad_idx])
    sd_p = jnp.concatenate([sd, pad_idx])
    return n_p, e_pad, ss_p, sd_p


def _graph_stage(h, inz, iz, lam, deg, n_p, e_pad, ss_p, sd_p,
                 W_c1, b_c1, W_c2, b_c2, W_sg1, b_sg1, W_sg2, b_sg2,
                 W_lin3, b_lin3):
    n = inz.shape[0]
    D = h.shape[1]
    dinv = jnp.clip(deg[:n], 1.0, None) ** -0.5
    dinv_b = jnp.broadcast_to(dinv[:, None], (n, D))
    r = 2.0 / lam
    h_nz = h[inz]
    h_z = h[iz]

    sd2d = sd_p.reshape(-1, _EB)
    ss2d = ss_p.reshape(-1, _EB)
    spmm = _make_spmm_kernel(n_p, e_pad, D)

    def cheb(X0, xs, W, b):
        msg2 = spmm(xs, ss2d, sd2d)
        return _cheb_mm(X0, msg2[0], msg2[1], dinv_b,
                        W[:D] - W[D:], r * W[D:], b, n_p)

    xs0 = jnp.pad(h_nz * dinv[:, None], ((0, n_p - n), (0, 0)))
    h01, xs1 = cheb(h_nz, xs0, W_c1, b_c1)
    h11, _ = cheb(h01, xs1, W_c2, b_c2)
    top = _mm2(h01, h11, W_lin3[:D], W_lin3[D:], b_lin3)
    bot = _zbranch(h_z, W_sg1, b_sg1, W_sg2, b_sg2,
                   W_lin3[:D], W_lin3[D:], b_lin3)
    return jnp.concatenate([top, bot], axis=0)


def kernel(in_feat, W_lin, b_lin, W_lin2, b_lin2, W_sg1, b_sg1, W_sg2, b_sg2,
           W_lin3, b_lin3, W_lin4, b_lin4, W_c1, b_c1, W_c2, b_c2,
           sub_src0, sub_dst0, idx_nz0, idx_z0, lam0,
           sub_src1, sub_dst1, idx_nz1, idx_z1, lam1,
           sub_src2, sub_dst2, idx_nz2, idx_z2, lam2):
    graphs = [(sub_src0, sub_dst0, idx_nz0, idx_z0, lam0),
              (sub_src1, sub_dst1, idx_nz1, idx_z1, lam1),
              (sub_src2, sub_dst2, idx_nz2, idx_z2, lam2)]

    pads = [_pad_graph(ss, sd, inz.shape[0])
            for ss, sd, inz, iz, lam in graphs]
    n_ps = tuple(p[0] for p in pads)
    e_pads = tuple(p[1] for p in pads)

    degk = _make_deg_kernel(n_ps, e_pads)
    degs = degk(pads[0][3].reshape(-1, _EB), pads[1][3].reshape(-1, _EB),
                pads[2][3].reshape(-1, _EB))

    h = _mm(in_feat, W_lin, b_lin, act="leaky")
    h = _mm(h, W_lin2, b_lin2, act="leaky")

    hs = []
    for g, (ss, sd, inz, iz, lam) in enumerate(graphs):
        n_p, e_pad, ss_p, sd_p = pads[g]
        h = _graph_stage(h, inz, iz, lam, degs[g], n_p, e_pad, ss_p, sd_p,
                         W_c1, b_c1, W_c2, b_c2, W_sg1, b_sg1, W_sg2, b_sg2,
                         W_lin3, b_lin3)
        hs.append(h)

    NCo = W_lin4.shape[1]
    W4p = jnp.zeros((W_lin4.shape[0], 128), jnp.float32).at[:, :NCo].set(W_lin4)
    b4p = jnp.zeros((128,), jnp.float32).at[:NCo].set(b_lin4)
    out = _final3(hs[0], hs[1], hs[2], W4p, b4p)
    return out[:, :NCo]
